# Initial kernel scaffold; baseline (speedup 1.0000x reference)
import jax
import jax.numpy as jnp
from jax.experimental import pallas as pl


def kernel(x, e, edge_index, W_pe, b_pe, W_e, b_e, A1, A2, A3, U, V, bA, bU, gn, bnb, ge, be, W1, b1, W2, b2):
    raise NotImplementedError("write your pallas kernel here")


# speedup vs baseline: 1.9099x; 1.9099x over previous
import functools

import jax
import jax.numpy as jnp
from jax import lax
from jax.experimental import pallas as pl
from jax.experimental.pallas import tpu as pltpu
from jax.experimental.pallas import tpu_sc as plsc

N = 50000
E = 800000
H = 32
L = 4

NCORES = 2
NSUB = 16
NW = NCORES * NSUB
CH = 128
NCHUNKS = E // CH
ROWS_PER_TILE = N // NSUB

_mesh = plsc.VectorSubcoreMesh(
    core_axis_name="c", subcore_axis_name="s", num_cores=NCORES,
    num_subcores=NSUB)


@functools.partial(
    pl.kernel,
    out_type=(
        jax.ShapeDtypeStruct((E, H), jnp.float32),
        jax.ShapeDtypeStruct((E, H), jnp.float32),
        jax.ShapeDtypeStruct((E, H), jnp.float32),
    ),
    mesh=_mesh,
    compiler_params=pltpu.CompilerParams(use_tc_tiling_on_sc=False),
    scratch_types=[
        pltpu.VMEM((CH,), jnp.int32),
        pltpu.VMEM((CH,), jnp.int32),
        pltpu.VMEM((CH, H), jnp.float32),
        pltpu.VMEM((CH, H), jnp.float32),
        pltpu.VMEM((CH, H), jnp.float32),
        pltpu.SemaphoreType.DMA,
    ],
)
def _sc_gather3(p1, p2, pv, dst2d, src2d, g1, g2, gv,
                idxd, idxs, b1, b2, b3, sem):
    c = lax.axis_index("c")
    s = lax.axis_index("s")
    w = s * NCORES + c
    nbase = NCHUNKS // NW
    rem = NCHUNKS % NW
    start = w * nbase + jnp.minimum(w, rem)
    cnt = nbase + jnp.where(w < rem, 1, 0)

    def body(j, carry):
        ch = start + j
        pltpu.sync_copy(dst2d.at[ch], idxd)
        pltpu.sync_copy(src2d.at[ch], idxs)
        cp1 = pltpu.async_copy(p1.at[idxd], b1, sem)
        cp2 = pltpu.async_copy(p2.at[idxs], b2, sem)
        cp3 = pltpu.async_copy(pv.at[idxs], b3, sem)
        cp1.wait()
        cp2.wait()
        cp3.wait()
        row = ch * CH
        pltpu.sync_copy(b1, g1.at[pl.ds(row, CH)])
        pltpu.sync_copy(b2, g2.at[pl.ds(row, CH)])
        pltpu.sync_copy(b3, gv.at[pl.ds(row, CH)])
        return carry

    lax.fori_loop(0, cnt, body, 0)


@functools.partial(
    pl.kernel,
    out_type=(
        jax.ShapeDtypeStruct((E, H), jnp.float32),
        jax.ShapeDtypeStruct((E, H), jnp.float32),
    ),
    mesh=_mesh,
    compiler_params=pltpu.CompilerParams(use_tc_tiling_on_sc=False),
    scratch_types=[
        pltpu.VMEM((CH,), jnp.int32),
        pltpu.VMEM((CH,), jnp.int32),
        pltpu.VMEM((CH, H), jnp.float32),
        pltpu.VMEM((CH, H), jnp.float32),
        pltpu.SemaphoreType.DMA,
    ],
)
def _sc_gather2(qs, qd, dst2d, src2d, gs, gd, idxd, idxs, b1, b2, sem):
    c = lax.axis_index("c")
    s = lax.axis_index("s")
    w = s * NCORES + c
    nbase = NCHUNKS // NW
    rem = NCHUNKS % NW
    start = w * nbase + jnp.minimum(w, rem)
    cnt = nbase + jnp.where(w < rem, 1, 0)

    def body(j, carry):
        ch = start + j
        pltpu.sync_copy(dst2d.at[ch], idxd)
        pltpu.sync_copy(src2d.at[ch], idxs)
        cp1 = pltpu.async_copy(qs.at[idxs], b1, sem)
        cp2 = pltpu.async_copy(qd.at[idxd], b2, sem)
        cp1.wait()
        cp2.wait()
        row = ch * CH
        pltpu.sync_copy(b1, gs.at[pl.ds(row, CH)])
        pltpu.sync_copy(b2, gd.at[pl.ds(row, CH)])
        return carry

    lax.fori_loop(0, cnt, body, 0)


@functools.partial(
    pl.kernel,
    out_type=jax.ShapeDtypeStruct((2 * N, H), jnp.float32),
    mesh=_mesh,
    compiler_params=pltpu.CompilerParams(use_tc_tiling_on_sc=False),
    scratch_types=[
        pltpu.VMEM_SHARED((N, H), jnp.float32),
        pltpu.VMEM((1, CH), jnp.int32),
        pltpu.VMEM((CH, H), jnp.float32),
    ],
)
def _sc_scatter2(msg, sig, dst2d, zeros, out, acc, idx2, vals):
    c = lax.axis_index("c")
    s = lax.axis_index("s")
    pltpu.sync_copy(zeros.at[pl.ds(0, ROWS_PER_TILE)],
                    acc.at[pl.ds(s * ROWS_PER_TILE, ROWS_PER_TILE)])
    plsc.subcore_barrier()

    nbase = NCHUNKS // NSUB
    rem = NCHUNKS % NSUB
    start = s * nbase + jnp.minimum(s, rem)
    cnt = nbase + jnp.where(s < rem, 1, 0)

    def make_body(vals_hbm):
        def body(j, carry):
            ch = start + j
            pltpu.sync_copy(dst2d.at[ch], idx2.at[0])
            pltpu.sync_copy(vals_hbm.at[pl.ds(ch * CH, CH)], vals)
            pltpu.sync_copy(vals, acc.at[idx2.at[0]], add=True)
            return carry
        return body

    @pl.when(c == 0)
    def _():
        lax.fori_loop(0, cnt, make_body(msg), 0)

    @pl.when(c == 1)
    def _():
        lax.fori_loop(0, cnt, make_body(sig), 0)

    plsc.subcore_barrier()
    pltpu.sync_copy(acc.at[pl.ds(s * ROWS_PER_TILE, ROWS_PER_TILE)],
                    out.at[pl.ds(c * N + s * ROWS_PER_TILE, ROWS_PER_TILE)])


def _bcast_init(v, w_row, b_row, rows, blk):
    def body(v_ref, w_ref, b_ref, o_ref):
        o_ref[...] = v_ref[...] * w_ref[...] + b_ref[...]

    grid = rows // blk
    return pl.pallas_call(
        body,
        grid=(grid,),
        in_specs=[
            pl.BlockSpec((blk, 1), lambda i: (i, 0)),
            pl.BlockSpec((1, H), lambda i: (0, 0)),
            pl.BlockSpec((1, H), lambda i: (0, 0)),
        ],
        out_specs=pl.BlockSpec((blk, H), lambda i: (i, 0)),
        out_shape=jax.ShapeDtypeStruct((rows, H), jnp.float32),
    )(v, w_row, b_row)


def _node_pre(h, a1, a2, vv, u, bu_row):
    def body(h_ref, a1_ref, a2_ref, v_ref, u_ref, bu_ref,
             p1_ref, p2_ref, pv_ref, hu_ref):
        hb = h_ref[...]
        p1_ref[...] = jnp.dot(hb, a1_ref[...], preferred_element_type=jnp.float32)
        p2_ref[...] = jnp.dot(hb, a2_ref[...], preferred_element_type=jnp.float32)
        pv_ref[...] = jnp.dot(hb, v_ref[...], preferred_element_type=jnp.float32)
        hu_ref[...] = jnp.dot(hb, u_ref[...], preferred_element_type=jnp.float32) + bu_ref[...]

    blk = 5000
    grid = N // blk
    spec = pl.BlockSpec((blk, H), lambda i: (i, 0))
    wspec = pl.BlockSpec((H, H), lambda i: (0, 0))
    return pl.pallas_call(
        body,
        grid=(grid,),
        in_specs=[spec, wspec, wspec, wspec, wspec,
                  pl.BlockSpec((1, H), lambda i: (0, 0))],
        out_specs=[spec, spec, spec, spec],
        out_shape=[jax.ShapeDtypeStruct((N, H), jnp.float32)] * 4,
    )(h, a1, a2, vv, u, bu_row)


def _edge_main(ef, g1, g2, gv, a3, ba_row):
    def body(ef_ref, g1_ref, g2_ref, gv_ref, a3_ref, ba_ref,
             eh_ref, sg_ref, msg_ref, st_ref):
        t = jnp.dot(ef_ref[...], a3_ref[...], preferred_element_type=jnp.float32)
        eh = g1_ref[...] + g2_ref[...] + t + ba_ref[...]
        eh_ref[...] = eh
        sg = jax.nn.sigmoid(eh)
        sg_ref[...] = sg
        msg_ref[...] = sg * gv_ref[...]
        s1 = jnp.sum(eh, axis=0, keepdims=True)
        s2 = jnp.sum(eh * eh, axis=0, keepdims=True)
        blk_stats = jnp.concatenate(
            [s1, s2, jnp.zeros((6, H), jnp.float32)], axis=0)

        @pl.when(pl.program_id(0) == 0)
        def _():
            st_ref[...] = jnp.zeros((8, H), jnp.float32)

        st_ref[...] += blk_stats

    blk = 4000
    grid = E // blk
    spec = pl.BlockSpec((blk, H), lambda i: (i, 0))
    return pl.pallas_call(
        body,
        grid=(grid,),
        in_specs=[spec, spec, spec, spec,
                  pl.BlockSpec((H, H), lambda i: (0, 0)),
                  pl.BlockSpec((1, H), lambda i: (0, 0))],
        out_specs=[spec, spec, spec,
                   pl.BlockSpec((8, H), lambda i: (0, 0))],
        out_shape=[
            jax.ShapeDtypeStruct((E, H), jnp.float32),
            jax.ShapeDtypeStruct((E, H), jnp.float32),
            jax.ShapeDtypeStruct((E, H), jnp.float32),
            jax.ShapeDtypeStruct((8, H), jnp.float32),
        ],
    )(ef, g1, g2, gv, a3, ba_row)


def _ef_update(ef, eh, st, ge_row, be_row):
    def body(ef_ref, eh_ref, st_ref, g_ref, b_ref, o_ref):
        mu = st_ref[0, :] / E
        var = st_ref[1, :] / E - mu * mu
        inv = lax.rsqrt(var + 1e-5)
        bn = g_ref[...] * (eh_ref[...] - mu[None, :]) * inv[None, :] + b_ref[...]
        o_ref[...] = ef_ref[...] + jnp.maximum(bn, 0.0)

    blk = 4000
    grid = E // blk
    spec = pl.BlockSpec((blk, H), lambda i: (i, 0))
    return pl.pallas_call(
        body,
        grid=(grid,),
        in_specs=[spec, spec,
                  pl.BlockSpec((8, H), lambda i: (0, 0)),
                  pl.BlockSpec((1, H), lambda i: (0, 0)),
                  pl.BlockSpec((1, H), lambda i: (0, 0))],
        out_specs=spec,
        out_shape=jax.ShapeDtypeStruct((E, H), jnp.float32),
    )(ef, eh, st, ge_row, be_row)


def _node_hhat(hu, aggden):
    def body(hu_ref, agg_ref, den_ref, hh_ref, st_ref):
        hh = hu_ref[...] + agg_ref[...] / (den_ref[...] + 1e-6)
        hh_ref[...] = hh
        s1 = jnp.sum(hh, axis=0, keepdims=True)
        s2 = jnp.sum(hh * hh, axis=0, keepdims=True)
        blk_stats = jnp.concatenate(
            [s1, s2, jnp.zeros((6, H), jnp.float32)], axis=0)

        @pl.when(pl.program_id(0) == 0)
        def _():
            st_ref[...] = jnp.zeros((8, H), jnp.float32)

        st_ref[...] += blk_stats

    blk = 5000
    grid = N // blk
    spec = pl.BlockSpec((blk, H), lambda i: (i, 0))
    return pl.pallas_call(
        body,
        grid=(grid,),
        in_specs=[spec, spec,
                  pl.BlockSpec((blk, H), lambda i: (i + N // blk, 0))],
        out_specs=[spec, pl.BlockSpec((8, H), lambda i: (0, 0))],
        out_shape=[
            jax.ShapeDtypeStruct((N, H), jnp.float32),
            jax.ShapeDtypeStruct((8, H), jnp.float32),
        ],
    )(hu, aggden, aggden)


def _node_apply(h, hh, st, gn_row, bnb_row):
    def body(h_ref, hh_ref, st_ref, g_ref, b_ref, o_ref):
        mu = st_ref[0, :] / N
        var = st_ref[1, :] / N - mu * mu
        inv = lax.rsqrt(var + 1e-5)
        bn = g_ref[...] * (hh_ref[...] - mu[None, :]) * inv[None, :] + b_ref[...]
        o_ref[...] = h_ref[...] + jnp.maximum(bn, 0.0)

    blk = 5000
    grid = N // blk
    spec = pl.BlockSpec((blk, H), lambda i: (i, 0))
    return pl.pallas_call(
        body,
        grid=(grid,),
        in_specs=[spec, spec,
                  pl.BlockSpec((8, H), lambda i: (0, 0)),
                  pl.BlockSpec((1, H), lambda i: (0, 0)),
                  pl.BlockSpec((1, H), lambda i: (0, 0))],
        out_specs=spec,
        out_shape=jax.ShapeDtypeStruct((N, H), jnp.float32),
    )(h, hh, st, gn_row, bnb_row)


def _node_final(h, w1a, w1b):
    def body(h_ref, wa_ref, wb_ref, qs_ref, qd_ref):
        hb = h_ref[...]
        qs_ref[...] = jnp.dot(hb, wa_ref[...], preferred_element_type=jnp.float32)
        qd_ref[...] = jnp.dot(hb, wb_ref[...], preferred_element_type=jnp.float32)

    blk = 5000
    grid = N // blk
    spec = pl.BlockSpec((blk, H), lambda i: (i, 0))
    wspec = pl.BlockSpec((H, H), lambda i: (0, 0))
    return pl.pallas_call(
        body,
        grid=(grid,),
        in_specs=[spec, wspec, wspec],
        out_specs=[spec, spec],
        out_shape=[jax.ShapeDtypeStruct((N, H), jnp.float32)] * 2,
    )(h, w1a, w1b)


def _final(gs, gd, ef, w1c, b1_row, w2, b2_row):
    def body(gs_ref, gd_ref, ef_ref, wc_ref, b1_ref, w2_ref, b2_ref, o_ref):
        t = jnp.dot(ef_ref[...], wc_ref[...], preferred_element_type=jnp.float32)
        z1 = jnp.maximum(gs_ref[...] + gd_ref[...] + t + b1_ref[...], 0.0)
        o_ref[...] = jnp.dot(z1, w2_ref[...], preferred_element_type=jnp.float32) + b2_ref[...]

    blk = 4000
    grid = E // blk
    spec = pl.BlockSpec((blk, H), lambda i: (i, 0))
    return pl.pallas_call(
        body,
        grid=(grid,),
        in_specs=[spec, spec, spec,
                  pl.BlockSpec((H, H), lambda i: (0, 0)),
                  pl.BlockSpec((1, H), lambda i: (0, 0)),
                  pl.BlockSpec((H, 1), lambda i: (0, 0)),
                  pl.BlockSpec((1, 1), lambda i: (0, 0))],
        out_specs=pl.BlockSpec((blk, 1), lambda i: (i, 0)),
        out_shape=jax.ShapeDtypeStruct((E, 1), jnp.float32),
    )(gs, gd, ef, w1c, b1_row, w2, b2_row)


def kernel(x, e, edge_index, W_pe, b_pe, W_e, b_e, A1, A2, A3, U, V,
           bA, bU, gn, bnb, ge, be, W1, b1, W2, b2):
    src2d = edge_index[0].reshape(NCHUNKS, CH)
    dst2d = edge_index[1].reshape(NCHUNKS, CH)
    zeros = jnp.zeros((ROWS_PER_TILE, H), jnp.float32)

    h = _bcast_init(x, W_pe, b_pe.reshape(1, H), N, 5000)
    ef = _bcast_init(e, W_e, b_e.reshape(1, H), E, 4000)

    for l in range(L):
        p1, p2, pv, hu = _node_pre(h, A1[l], A2[l], V[l], U[l],
                                   bU[l].reshape(1, H))
        g1, g2, gv = _sc_gather3(p1, p2, pv, dst2d, src2d)
        eh, sg, msg, st = _edge_main(ef, g1, g2, gv, A3[l],
                                     bA[l].reshape(1, H))
        aggden = _sc_scatter2(msg, sg, dst2d, zeros)
        hh, hst = _node_hhat(hu, aggden)
        h = _node_apply(h, hh, hst, gn[l].reshape(1, H), bnb[l].reshape(1, H))
        ef = _ef_update(ef, eh, st, ge[l].reshape(1, H), be[l].reshape(1, H))

    qs, qd = _node_final(h, W1[:H], W1[H:2 * H])
    gs, gd = _sc_gather2(qs, qd, dst2d, src2d)
    return _final(gs, gd, ef, W1[2 * H:], b1.reshape(1, H), W2,
                  b2.reshape(1, 1))


# speedup vs baseline: 3.0235x; 1.5831x over previous
import functools

import jax
import jax.numpy as jnp
from jax import lax
from jax.experimental import pallas as pl
from jax.experimental.pallas import tpu as pltpu
from jax.experimental.pallas import tpu_sc as plsc

N = 50000
E = 800000
H = 32
L = 4

NCORES = 2
NSUB = 16
NW = NCORES * NSUB
CH = 128
NCHUNKS = E // CH
KSG = 5
SG = KSG * CH
NSG = E // SG
ROWS_PER_TILE = N // NSUB

_mesh = plsc.VectorSubcoreMesh(
    core_axis_name="c", subcore_axis_name="s", num_cores=NCORES,
    num_subcores=NSUB)

_SC_PARAMS = pltpu.CompilerParams(use_tc_tiling_on_sc=False)


@functools.partial(
    pl.kernel,
    out_type=(
        jax.ShapeDtypeStruct((E, H), jnp.float32),
        jax.ShapeDtypeStruct((E, H), jnp.float32),
        jax.ShapeDtypeStruct((E, H), jnp.float32),
        jax.ShapeDtypeStruct((NW, 64), jnp.float32),
    ),
    mesh=_mesh,
    compiler_params=_SC_PARAMS,
    scratch_types=[
        pltpu.VMEM((KSG, CH), jnp.int32),
        pltpu.VMEM((KSG, CH), jnp.int32),
        pltpu.VMEM((SG, H), jnp.float32),
        pltpu.VMEM((SG, H), jnp.float32),
        pltpu.VMEM((SG, H), jnp.float32),
        pltpu.VMEM((SG, H), jnp.float32),
        pltpu.VMEM((64,), jnp.float32),
        pltpu.SemaphoreType.DMA,
        pltpu.SemaphoreType.DMA,
        pltpu.SemaphoreType.DMA,
    ],
)
def _sc_edge(p1, p2, pv, t, dst2d, src2d, ehat, sig, msg, stats,
             idxd, idxs, g1b, g2b, gvb, tb, sbuf, sem_g, sem_t, sem_w):
    c = lax.axis_index("c")
    s = lax.axis_index("s")
    w = s * NCORES + c
    nbase = NSG // NW
    rem = NSG % NW
    sg0 = w * nbase + jnp.minimum(w, rem)
    nsg = nbase + jnp.where(w < rem, 1, 0)

    def sg_body(i, st):
        sgi = sg0 + i
        crow = sgi * KSG
        row0 = sgi * SG
        pltpu.sync_copy(dst2d.at[pl.ds(crow, KSG)], idxd)
        pltpu.sync_copy(src2d.at[pl.ds(crow, KSG)], idxs)
        tcp = pltpu.async_copy(t.at[pl.ds(row0, SG)], tb, sem_t)
        cps = []
        for k in range(KSG):
            cps.append(pltpu.async_copy(
                p1.at[idxd.at[k]], g1b.at[pl.ds(k * CH, CH)], sem_g))
            cps.append(pltpu.async_copy(
                p2.at[idxs.at[k]], g2b.at[pl.ds(k * CH, CH)], sem_g))
            cps.append(pltpu.async_copy(
                pv.at[idxs.at[k]], gvb.at[pl.ds(k * CH, CH)], sem_g))
        tcp.wait()
        for cp in cps:
            cp.wait()

        def row(r, st2):
            a0, a1, q0, q1 = st2
            e0 = g1b[r, pl.ds(0, 16)] + g2b[r, pl.ds(0, 16)] + tb[r, pl.ds(0, 16)]
            e1 = g1b[r, pl.ds(16, 16)] + g2b[r, pl.ds(16, 16)] + tb[r, pl.ds(16, 16)]
            s0 = 1.0 / (1.0 + jnp.exp(-e0))
            s1 = 1.0 / (1.0 + jnp.exp(-e1))
            g1b[r, pl.ds(0, 16)] = e0
            g1b[r, pl.ds(16, 16)] = e1
            g2b[r, pl.ds(0, 16)] = s0
            g2b[r, pl.ds(16, 16)] = s1
            gvb[r, pl.ds(0, 16)] = s0 * gvb[r, pl.ds(0, 16)]
            gvb[r, pl.ds(16, 16)] = s1 * gvb[r, pl.ds(16, 16)]
            return (a0 + e0, a1 + e1, q0 + e0 * e0, q1 + e1 * e1)

        st = lax.fori_loop(0, SG, row, st)
        w1 = pltpu.async_copy(g1b, ehat.at[pl.ds(row0, SG)], sem_w)
        w2 = pltpu.async_copy(g2b, sig.at[pl.ds(row0, SG)], sem_w)
        w3 = pltpu.async_copy(gvb, msg.at[pl.ds(row0, SG)], sem_w)
        w1.wait()
        w2.wait()
        w3.wait()
        return st

    z = jnp.zeros((16,), jnp.float32)
    a0, a1, q0, q1 = lax.fori_loop(0, nsg, sg_body, (z, z, z, z))
    sbuf[pl.ds(0, 16)] = a0
    sbuf[pl.ds(16, 16)] = a1
    sbuf[pl.ds(32, 16)] = q0
    sbuf[pl.ds(48, 16)] = q1
    pltpu.sync_copy(sbuf, stats.at[w])


@functools.partial(
    pl.kernel,
    out_type=jax.ShapeDtypeStruct((2 * N, H), jnp.float32),
    mesh=_mesh,
    compiler_params=_SC_PARAMS,
    scratch_types=[
        pltpu.VMEM_SHARED((N, H), jnp.float32),
        pltpu.VMEM((1, CH), jnp.int32),
        pltpu.VMEM((CH, H), jnp.float32),
    ],
)
def _sc_scatter2(msg, sig, dst2d, zeros, out, acc, idx2, vals):
    c = lax.axis_index("c")
    s = lax.axis_index("s")
    pltpu.sync_copy(zeros.at[pl.ds(0, ROWS_PER_TILE)],
                    acc.at[pl.ds(s * ROWS_PER_TILE, ROWS_PER_TILE)])
    plsc.subcore_barrier()

    nbase = NCHUNKS // NSUB
    rem = NCHUNKS % NSUB
    start = s * nbase + jnp.minimum(s, rem)
    cnt = nbase + jnp.where(s < rem, 1, 0)

    def make_body(vals_hbm):
        def body(j, carry):
            ch = start + j
            pltpu.sync_copy(dst2d.at[ch], idx2.at[0])
            pltpu.sync_copy(vals_hbm.at[pl.ds(ch * CH, CH)], vals)
            pltpu.sync_copy(vals, acc.at[idx2.at[0]], add=True)
            return carry
        return body

    @pl.when(c == 0)
    def _():
        lax.fori_loop(0, cnt, make_body(msg), 0)

    @pl.when(c == 1)
    def _():
        lax.fori_loop(0, cnt, make_body(sig), 0)

    plsc.subcore_barrier()
    pltpu.sync_copy(acc.at[pl.ds(s * ROWS_PER_TILE, ROWS_PER_TILE)],
                    out.at[pl.ds(c * N + s * ROWS_PER_TILE, ROWS_PER_TILE)])


@functools.partial(
    pl.kernel,
    out_type=(
        jax.ShapeDtypeStruct((E, H), jnp.float32),
        jax.ShapeDtypeStruct((E, H), jnp.float32),
    ),
    mesh=_mesh,
    compiler_params=_SC_PARAMS,
    scratch_types=[
        pltpu.VMEM((KSG, CH), jnp.int32),
        pltpu.VMEM((KSG, CH), jnp.int32),
        pltpu.VMEM((SG, H), jnp.float32),
        pltpu.VMEM((SG, H), jnp.float32),
        pltpu.SemaphoreType.DMA,
    ],
)
def _sc_gather2(qs, qd, dst2d, src2d, gs, gd, idxd, idxs, b1, b2, sem):
    c = lax.axis_index("c")
    s = lax.axis_index("s")
    w = s * NCORES + c
    nbase = NSG // NW
    rem = NSG % NW
    sg0 = w * nbase + jnp.minimum(w, rem)
    nsg = nbase + jnp.where(w < rem, 1, 0)

    def body(i, carry):
        sgi = sg0 + i
        crow = sgi * KSG
        row0 = sgi * SG
        pltpu.sync_copy(dst2d.at[pl.ds(crow, KSG)], idxd)
        pltpu.sync_copy(src2d.at[pl.ds(crow, KSG)], idxs)
        cps = []
        for k in range(KSG):
            cps.append(pltpu.async_copy(
                qs.at[idxs.at[k]], b1.at[pl.ds(k * CH, CH)], sem))
            cps.append(pltpu.async_copy(
                qd.at[idxd.at[k]], b2.at[pl.ds(k * CH, CH)], sem))
        for cp in cps:
            cp.wait()
        c1 = pltpu.async_copy(b1, gs.at[pl.ds(row0, SG)], sem)
        c2 = pltpu.async_copy(b2, gd.at[pl.ds(row0, SG)], sem)
        c1.wait()
        c2.wait()
        return carry

    lax.fori_loop(0, nsg, body, 0)


_BLK_E = 4000
_BLK_N = 5000


def _stats_mean_var(st_ref, count):
    ssum = jnp.sum(st_ref[...], axis=0)
    mu = jnp.concatenate([ssum[0:16], ssum[16:32]]) / count
    msq = jnp.concatenate([ssum[32:48], ssum[48:64]]) / count
    var = msq - mu * mu
    return mu, lax.rsqrt(var + 1e-5)


def _bcast_init(v, w_row, b_row, rows, blk):
    def body(v_ref, w_ref, b_ref, o_ref):
        o_ref[...] = v_ref[...] * w_ref[...] + b_ref[...]

    return pl.pallas_call(
        body,
        grid=(rows // blk,),
        in_specs=[
            pl.BlockSpec((blk, 1), lambda i: (i, 0)),
            pl.BlockSpec((1, H), lambda i: (0, 0)),
            pl.BlockSpec((1, H), lambda i: (0, 0)),
        ],
        out_specs=pl.BlockSpec((blk, H), lambda i: (i, 0)),
        out_shape=jax.ShapeDtypeStruct((rows, H), jnp.float32),
    )(v, w_row, b_row)


def _eft0(e, w_row, b_row, a3, ba_row):
    def body(e_ref, w_ref, b_ref, a3_ref, ba_ref, ef_ref, t_ref):
        ef = e_ref[...] * w_ref[...] + b_ref[...]
        ef_ref[...] = ef
        t_ref[...] = jnp.dot(ef, a3_ref[...],
                             preferred_element_type=jnp.float32) + ba_ref[...]

    spec = pl.BlockSpec((_BLK_E, H), lambda i: (i, 0))
    return pl.pallas_call(
        body,
        grid=(E // _BLK_E,),
        in_specs=[pl.BlockSpec((_BLK_E, 1), lambda i: (i, 0)),
                  pl.BlockSpec((1, H), lambda i: (0, 0)),
                  pl.BlockSpec((1, H), lambda i: (0, 0)),
                  pl.BlockSpec((H, H), lambda i: (0, 0)),
                  pl.BlockSpec((1, H), lambda i: (0, 0))],
        out_specs=[spec, spec],
        out_shape=[jax.ShapeDtypeStruct((E, H), jnp.float32)] * 2,
    )(e, w_row, b_row, a3, ba_row)


def _eft(ef, eh, st, ge_row, be_row, a3, ba_row):
    def body(ef_ref, eh_ref, st_ref, g_ref, b_ref, a3_ref, ba_ref,
             ef_o, t_o):
        mu, inv = _stats_mean_var(st_ref, float(E))
        bn = g_ref[...] * (eh_ref[...] - mu[None, :]) * inv[None, :] + b_ref[...]
        ef_new = ef_ref[...] + jnp.maximum(bn, 0.0)
        ef_o[...] = ef_new
        t_o[...] = jnp.dot(ef_new, a3_ref[...],
                           preferred_element_type=jnp.float32) + ba_ref[...]

    spec = pl.BlockSpec((_BLK_E, H), lambda i: (i, 0))
    row = pl.BlockSpec((1, H), lambda i: (0, 0))
    return pl.pallas_call(
        body,
        grid=(E // _BLK_E,),
        in_specs=[spec, spec,
                  pl.BlockSpec((NW, 64), lambda i: (0, 0)),
                  row, row,
                  pl.BlockSpec((H, H), lambda i: (0, 0)), row],
        out_specs=[spec, spec],
        out_shape=[jax.ShapeDtypeStruct((E, H), jnp.float32)] * 2,
    )(ef, eh, st, ge_row, be_row, a3, ba_row)


def _node_pre(h, a1, a2, vv, u, bu_row):
    def body(h_ref, a1_ref, a2_ref, v_ref, u_ref, bu_ref,
             p1_ref, p2_ref, pv_ref, hu_ref):
        hb = h_ref[...]
        p1_ref[...] = jnp.dot(hb, a1_ref[...], preferred_element_type=jnp.float32)
        p2_ref[...] = jnp.dot(hb, a2_ref[...], preferred_element_type=jnp.float32)
        pv_ref[...] = jnp.dot(hb, v_ref[...], preferred_element_type=jnp.float32)
        hu_ref[...] = jnp.dot(hb, u_ref[...], preferred_element_type=jnp.float32) + bu_ref[...]

    spec = pl.BlockSpec((_BLK_N, H), lambda i: (i, 0))
    wspec = pl.BlockSpec((H, H), lambda i: (0, 0))
    return pl.pallas_call(
        body,
        grid=(N // _BLK_N,),
        in_specs=[spec, wspec, wspec, wspec, wspec,
                  pl.BlockSpec((1, H), lambda i: (0, 0))],
        out_specs=[spec, spec, spec, spec],
        out_shape=[jax.ShapeDtypeStruct((N, H), jnp.float32)] * 4,
    )(h, a1, a2, vv, u, bu_row)


def _node_hhat(hu, aggden):
    def body(hu_ref, agg_ref, den_ref, hh_ref, st_ref):
        hh = hu_ref[...] + agg_ref[...] / (den_ref[...] + 1e-6)
        hh_ref[...] = hh
        s1 = jnp.sum(hh, axis=0, keepdims=True)
        s2 = jnp.sum(hh * hh, axis=0, keepdims=True)
        blk_stats = jnp.concatenate(
            [s1, s2, jnp.zeros((6, H), jnp.float32)], axis=0)

        @pl.when(pl.program_id(0) == 0)
        def _():
            st_ref[...] = jnp.zeros((8, H), jnp.float32)

        st_ref[...] += blk_stats

    spec = pl.BlockSpec((_BLK_N, H), lambda i: (i, 0))
    return pl.pallas_call(
        body,
        grid=(N // _BLK_N,),
        in_specs=[spec, spec,
                  pl.BlockSpec((_BLK_N, H), lambda i: (i + N // _BLK_N, 0))],
        out_specs=[spec, pl.BlockSpec((8, H), lambda i: (0, 0))],
        out_shape=[
            jax.ShapeDtypeStruct((N, H), jnp.float32),
            jax.ShapeDtypeStruct((8, H), jnp.float32),
        ],
    )(hu, aggden, aggden)


def _node_apply(h, hh, st, gn_row, bnb_row):
    def body(h_ref, hh_ref, st_ref, g_ref, b_ref, o_ref):
        mu = st_ref[0, :] / N
        var = st_ref[1, :] / N - mu * mu
        inv = lax.rsqrt(var + 1e-5)
        bn = g_ref[...] * (hh_ref[...] - mu[None, :]) * inv[None, :] + b_ref[...]
        o_ref[...] = h_ref[...] + jnp.maximum(bn, 0.0)

    spec = pl.BlockSpec((_BLK_N, H), lambda i: (i, 0))
    return pl.pallas_call(
        body,
        grid=(N // _BLK_N,),
        in_specs=[spec, spec,
                  pl.BlockSpec((8, H), lambda i: (0, 0)),
                  pl.BlockSpec((1, H), lambda i: (0, 0)),
                  pl.BlockSpec((1, H), lambda i: (0, 0))],
        out_specs=spec,
        out_shape=jax.ShapeDtypeStruct((N, H), jnp.float32),
    )(h, hh, st, gn_row, bnb_row)


def _node_final(h, w1a, w1b):
    def body(h_ref, wa_ref, wb_ref, qs_ref, qd_ref):
        hb = h_ref[...]
        qs_ref[...] = jnp.dot(hb, wa_ref[...], preferred_element_type=jnp.float32)
        qd_ref[...] = jnp.dot(hb, wb_ref[...], preferred_element_type=jnp.float32)

    spec = pl.BlockSpec((_BLK_N, H), lambda i: (i, 0))
    wspec = pl.BlockSpec((H, H), lambda i: (0, 0))
    return pl.pallas_call(
        body,
        grid=(N // _BLK_N,),
        in_specs=[spec, wspec, wspec],
        out_specs=[spec, spec],
        out_shape=[jax.ShapeDtypeStruct((N, H), jnp.float32)] * 2,
    )(h, w1a, w1b)


def _final(gs, gd, ef, eh, st, ge_row, be_row, w1c, b1_row, w2, b2_row):
    def body(gs_ref, gd_ref, ef_ref, eh_ref, st_ref, g_ref, b_ref,
             wc_ref, b1_ref, w2_ref, b2_ref, o_ref):
        mu, inv = _stats_mean_var(st_ref, float(E))
        bn = g_ref[...] * (eh_ref[...] - mu[None, :]) * inv[None, :] + b_ref[...]
        ef_l = ef_ref[...] + jnp.maximum(bn, 0.0)
        t = jnp.dot(ef_l, wc_ref[...], preferred_element_type=jnp.float32)
        z1 = jnp.maximum(gs_ref[...] + gd_ref[...] + t + b1_ref[...], 0.0)
        o_ref[...] = jnp.dot(z1, w2_ref[...], preferred_element_type=jnp.float32) + b2_ref[...]

    spec = pl.BlockSpec((_BLK_E, H), lambda i: (i, 0))
    row = pl.BlockSpec((1, H), lambda i: (0, 0))
    return pl.pallas_call(
        body,
        grid=(E // _BLK_E,),
        in_specs=[spec, spec, spec, spec,
                  pl.BlockSpec((NW, 64), lambda i: (0, 0)),
                  row, row,
                  pl.BlockSpec((H, H), lambda i: (0, 0)), row,
                  pl.BlockSpec((H, 1), lambda i: (0, 0)),
                  pl.BlockSpec((1, 1), lambda i: (0, 0))],
        out_specs=pl.BlockSpec((_BLK_E, 1), lambda i: (i, 0)),
        out_shape=jax.ShapeDtypeStruct((E, 1), jnp.float32),
    )(gs, gd, ef, eh, st, ge_row, be_row, w1c, b1_row, w2, b2_row)


def kernel(x, e, edge_index, W_pe, b_pe, W_e, b_e, A1, A2, A3, U, V,
           bA, bU, gn, bnb, ge, be, W1, b1, W2, b2):
    src2d = edge_index[0].reshape(NCHUNKS, CH)
    dst2d = edge_index[1].reshape(NCHUNKS, CH)
    zeros = jnp.zeros((ROWS_PER_TILE, H), jnp.float32)

    h = _bcast_init(x, W_pe, b_pe.reshape(1, H), N, _BLK_N)

    ef = None
    eh = None
    st = None
    for l in range(L):
        if l == 0:
            ef, t = _eft0(e, W_e, b_e.reshape(1, H), A3[0], bA[0].reshape(1, H))
        else:
            ef, t = _eft(ef, eh, st, ge[l - 1].reshape(1, H),
                         be[l - 1].reshape(1, H), A3[l], bA[l].reshape(1, H))
        p1, p2, pv, hu = _node_pre(h, A1[l], A2[l], V[l], U[l],
                                   bU[l].reshape(1, H))
        eh, sg, msg, st = _sc_edge(p1, p2, pv, t, dst2d, src2d)
        aggden = _sc_scatter2(msg, sg, dst2d, zeros)
        hh, hst = _node_hhat(hu, aggden)
        h = _node_apply(h, hh, hst, gn[l].reshape(1, H), bnb[l].reshape(1, H))

    qs, qd = _node_final(h, W1[:H], W1[H:2 * H])
    gs, gd = _sc_gather2(qs, qd, dst2d, src2d)
    return _final(gs, gd, ef, eh, st, ge[L - 1].reshape(1, H),
                  be[L - 1].reshape(1, H), W1[2 * H:], b1.reshape(1, H), W2,
                  b2.reshape(1, 1))


# speedup vs baseline: 3.1623x; 1.0459x over previous
import functools

import jax
import jax.numpy as jnp
from jax import lax
from jax.experimental import pallas as pl
from jax.experimental.pallas import tpu as pltpu
from jax.experimental.pallas import tpu_sc as plsc

N = 50000
E = 800000
H = 32
L = 4

NCORES = 2
NSUB = 16
NW = NCORES * NSUB
CH = 128
NCHUNKS = E // CH
KSG = 5
SG = KSG * CH
NSG = E // SG
ROWS_PER_TILE = N // NSUB

_mesh = plsc.VectorSubcoreMesh(
    core_axis_name="c", subcore_axis_name="s", num_cores=NCORES,
    num_subcores=NSUB)

_SC_PARAMS = pltpu.CompilerParams(use_tc_tiling_on_sc=False)


@functools.partial(
    pl.kernel,
    out_type=(
        jax.ShapeDtypeStruct((E, H), jnp.float32),
        jax.ShapeDtypeStruct((E, H), jnp.float32),
        jax.ShapeDtypeStruct((E, H), jnp.float32),
        jax.ShapeDtypeStruct((NW, 64), jnp.float32),
    ),
    mesh=_mesh,
    compiler_params=_SC_PARAMS,
    scratch_types=[
        pltpu.VMEM((KSG, CH), jnp.int32),
        pltpu.VMEM((KSG, CH), jnp.int32),
        pltpu.VMEM((SG, H), jnp.float32),
        pltpu.VMEM((SG, H), jnp.float32),
        pltpu.VMEM((SG, H), jnp.float32),
        pltpu.VMEM((SG, H), jnp.float32),
        pltpu.VMEM((64,), jnp.float32),
        pltpu.SemaphoreType.DMA,
        pltpu.SemaphoreType.DMA,
        pltpu.SemaphoreType.DMA,
    ],
)
def _sc_edge(p1, p2, pv, t, dst2d, src2d, ehat, sig, msg, stats,
             idxd, idxs, g1b, g2b, gvb, tb, sbuf, sem_g, sem_t, sem_w):
    c = lax.axis_index("c")
    s = lax.axis_index("s")
    w = s * NCORES + c
    nbase = NSG // NW
    rem = NSG % NW
    sg0 = w * nbase + jnp.minimum(w, rem)
    nsg = nbase + jnp.where(w < rem, 1, 0)

    def sg_body(i, st):
        sgi = sg0 + i
        crow = sgi * KSG
        row0 = sgi * SG
        pltpu.sync_copy(dst2d.at[pl.ds(crow, KSG)], idxd)
        pltpu.sync_copy(src2d.at[pl.ds(crow, KSG)], idxs)
        tcp = pltpu.async_copy(t.at[pl.ds(row0, SG)], tb, sem_t)
        cps = []
        for k in range(KSG):
            cps.append(pltpu.async_copy(
                p1.at[idxd.at[k]], g1b.at[pl.ds(k * CH, CH)], sem_g))
            cps.append(pltpu.async_copy(
                p2.at[idxs.at[k]], g2b.at[pl.ds(k * CH, CH)], sem_g))
            cps.append(pltpu.async_copy(
                pv.at[idxs.at[k]], gvb.at[pl.ds(k * CH, CH)], sem_g))
        tcp.wait()
        for cp in cps:
            cp.wait()

        def row(r, st2):
            a0, a1, q0, q1 = st2
            e0 = g1b[r, pl.ds(0, 16)] + g2b[r, pl.ds(0, 16)] + tb[r, pl.ds(0, 16)]
            e1 = g1b[r, pl.ds(16, 16)] + g2b[r, pl.ds(16, 16)] + tb[r, pl.ds(16, 16)]
            s0 = 1.0 / (1.0 + jnp.exp(-e0))
            s1 = 1.0 / (1.0 + jnp.exp(-e1))
            g1b[r, pl.ds(0, 16)] = e0
            g1b[r, pl.ds(16, 16)] = e1
            g2b[r, pl.ds(0, 16)] = s0
            g2b[r, pl.ds(16, 16)] = s1
            gvb[r, pl.ds(0, 16)] = s0 * gvb[r, pl.ds(0, 16)]
            gvb[r, pl.ds(16, 16)] = s1 * gvb[r, pl.ds(16, 16)]
            return (a0 + e0, a1 + e1, q0 + e0 * e0, q1 + e1 * e1)

        st = lax.fori_loop(0, SG, row, st)
        w1 = pltpu.async_copy(g1b, ehat.at[pl.ds(row0, SG)], sem_w)
        w2 = pltpu.async_copy(g2b, sig.at[pl.ds(row0, SG)], sem_w)
        w3 = pltpu.async_copy(gvb, msg.at[pl.ds(row0, SG)], sem_w)
        w1.wait()
        w2.wait()
        w3.wait()
        return st

    z = jnp.zeros((16,), jnp.float32)
    a0, a1, q0, q1 = lax.fori_loop(0, nsg, sg_body, (z, z, z, z))
    sbuf[pl.ds(0, 16)] = a0
    sbuf[pl.ds(16, 16)] = a1
    sbuf[pl.ds(32, 16)] = q0
    sbuf[pl.ds(48, 16)] = q1
    pltpu.sync_copy(sbuf, stats.at[w])


GRP = 3
NGRP = (NCHUNKS // NSUB) // GRP


@functools.partial(
    pl.kernel,
    out_type=jax.ShapeDtypeStruct((2 * N, H), jnp.float32),
    mesh=_mesh,
    compiler_params=_SC_PARAMS,
    scratch_types=[
        pltpu.VMEM_SHARED((N, H), jnp.float32),
        pltpu.VMEM((GRP, CH), jnp.int32),
        pltpu.VMEM((GRP, CH), jnp.int32),
        pltpu.VMEM((GRP * CH, H), jnp.float32),
        pltpu.VMEM((GRP * CH, H), jnp.float32),
        pltpu.SemaphoreType.DMA,
        pltpu.SemaphoreType.DMA,
    ],
)
def _sc_scatter2(msg, sig, dst2d, zeros, out, acc, idx0, idx1, val0, val1,
                 sem_l, sem_s):
    c = lax.axis_index("c")
    s = lax.axis_index("s")
    pltpu.sync_copy(zeros.at[pl.ds(0, ROWS_PER_TILE)],
                    acc.at[pl.ds(s * ROWS_PER_TILE, ROWS_PER_TILE)])
    plsc.subcore_barrier()

    nbase = NCHUNKS // NSUB
    rem = NCHUNKS % NSUB
    start = s * nbase + jnp.minimum(s, rem)
    cnt = nbase + jnp.where(s < rem, 1, 0)

    def pipe(vhbm):
        def load(g, idxs, vals):
            ch0 = start + g * GRP
            pltpu.async_copy(dst2d.at[pl.ds(ch0, GRP)], idxs, sem_l)
            pltpu.async_copy(vhbm.at[pl.ds(ch0 * CH, GRP * CH)], vals, sem_l)

        def wait_load(idxs, vals):
            pltpu.make_async_copy(dst2d.at[pl.ds(0, GRP)], idxs, sem_l).wait()
            pltpu.make_async_copy(vhbm.at[pl.ds(0, GRP * CH)], vals,
                                  sem_l).wait()

        def scat(idxs, vals):
            for k in range(GRP):
                pltpu.async_copy(vals.at[pl.ds(k * CH, CH)],
                                 acc.at[idxs.at[k]], sem_s, add=True)

        def wait_scat(vals):
            pltpu.make_async_copy(vhbm.at[pl.ds(0, GRP * CH)], vals,
                                  sem_s).wait()

        load(0, idx0, val0)

        def body(i, carry):
            ga = 2 * i
            wait_load(idx0, val0)

            @pl.when(i > 0)
            def _():
                wait_scat(val1)

            scat(idx0, val0)
            load(ga + 1, idx1, val1)
            wait_load(idx1, val1)
            wait_scat(val0)
            scat(idx1, val1)

            @pl.when(i < NGRP // 2 - 1)
            def _():
                load(ga + 2, idx0, val0)

            return carry

        lax.fori_loop(0, NGRP // 2, body, 0)
        wait_scat(val1)

        def tbody(j, carry):
            ch = start + NGRP * GRP + j
            pltpu.sync_copy(dst2d.at[ch], idx0.at[0])
            pltpu.sync_copy(vhbm.at[pl.ds(ch * CH, CH)],
                            val0.at[pl.ds(0, CH)])
            pltpu.sync_copy(val0.at[pl.ds(0, CH)], acc.at[idx0.at[0]],
                            add=True)
            return carry

        lax.fori_loop(0, cnt - NGRP * GRP, tbody, 0)

    @pl.when(c == 0)
    def _():
        pipe(msg)

    @pl.when(c == 1)
    def _():
        pipe(sig)

    plsc.subcore_barrier()
    pltpu.sync_copy(acc.at[pl.ds(s * ROWS_PER_TILE, ROWS_PER_TILE)],
                    out.at[pl.ds(c * N + s * ROWS_PER_TILE, ROWS_PER_TILE)])


@functools.partial(
    pl.kernel,
    out_type=(
        jax.ShapeDtypeStruct((E, H), jnp.float32),
        jax.ShapeDtypeStruct((E, H), jnp.float32),
    ),
    mesh=_mesh,
    compiler_params=_SC_PARAMS,
    scratch_types=[
        pltpu.VMEM((KSG, CH), jnp.int32),
        pltpu.VMEM((KSG, CH), jnp.int32),
        pltpu.VMEM((SG, H), jnp.float32),
        pltpu.VMEM((SG, H), jnp.float32),
        pltpu.SemaphoreType.DMA,
    ],
)
def _sc_gather2(qs, qd, dst2d, src2d, gs, gd, idxd, idxs, b1, b2, sem):
    c = lax.axis_index("c")
    s = lax.axis_index("s")
    w = s * NCORES + c
    nbase = NSG // NW
    rem = NSG % NW
    sg0 = w * nbase + jnp.minimum(w, rem)
    nsg = nbase + jnp.where(w < rem, 1, 0)

    def body(i, carry):
        sgi = sg0 + i
        crow = sgi * KSG
        row0 = sgi * SG
        pltpu.sync_copy(dst2d.at[pl.ds(crow, KSG)], idxd)
        pltpu.sync_copy(src2d.at[pl.ds(crow, KSG)], idxs)
        cps = []
        for k in range(KSG):
            cps.append(pltpu.async_copy(
                qs.at[idxs.at[k]], b1.at[pl.ds(k * CH, CH)], sem))
            cps.append(pltpu.async_copy(
                qd.at[idxd.at[k]], b2.at[pl.ds(k * CH, CH)], sem))
        for cp in cps:
            cp.wait()
        c1 = pltpu.async_copy(b1, gs.at[pl.ds(row0, SG)], sem)
        c2 = pltpu.async_copy(b2, gd.at[pl.ds(row0, SG)], sem)
        c1.wait()
        c2.wait()
        return carry

    lax.fori_loop(0, nsg, body, 0)


_BLK_E = 4000
_BLK_N = 5000


def _stats_mean_var(st_ref, count):
    ssum = jnp.sum(st_ref[...], axis=0)
    mu = jnp.concatenate([ssum[0:16], ssum[16:32]]) / count
    msq = jnp.concatenate([ssum[32:48], ssum[48:64]]) / count
    var = msq - mu * mu
    return mu, lax.rsqrt(var + 1e-5)


def _bcast_init(v, w_row, b_row, rows, blk):
    def body(v_ref, w_ref, b_ref, o_ref):
        o_ref[...] = v_ref[...] * w_ref[...] + b_ref[...]

    return pl.pallas_call(
        body,
        grid=(rows // blk,),
        in_specs=[
            pl.BlockSpec((blk, 1), lambda i: (i, 0)),
            pl.BlockSpec((1, H), lambda i: (0, 0)),
            pl.BlockSpec((1, H), lambda i: (0, 0)),
        ],
        out_specs=pl.BlockSpec((blk, H), lambda i: (i, 0)),
        out_shape=jax.ShapeDtypeStruct((rows, H), jnp.float32),
    )(v, w_row, b_row)


def _eft0(e, w_row, b_row, a3, ba_row):
    def body(e_ref, w_ref, b_ref, a3_ref, ba_ref, ef_ref, t_ref):
        ef = e_ref[...] * w_ref[...] + b_ref[...]
        ef_ref[...] = ef
        t_ref[...] = jnp.dot(ef, a3_ref[...],
                             preferred_element_type=jnp.float32) + ba_ref[...]

    spec = pl.BlockSpec((_BLK_E, H), lambda i: (i, 0))
    return pl.pallas_call(
        body,
        grid=(E // _BLK_E,),
        in_specs=[pl.BlockSpec((_BLK_E, 1), lambda i: (i, 0)),
                  pl.BlockSpec((1, H), lambda i: (0, 0)),
                  pl.BlockSpec((1, H), lambda i: (0, 0)),
                  pl.BlockSpec((H, H), lambda i: (0, 0)),
                  pl.BlockSpec((1, H), lambda i: (0, 0))],
        out_specs=[spec, spec],
        out_shape=[jax.ShapeDtypeStruct((E, H), jnp.float32)] * 2,
    )(e, w_row, b_row, a3, ba_row)


def _eft(ef, eh, st, ge_row, be_row, a3, ba_row):
    def body(ef_ref, eh_ref, st_ref, g_ref, b_ref, a3_ref, ba_ref,
             ef_o, t_o):
        mu, inv = _stats_mean_var(st_ref, float(E))
        bn = g_ref[...] * (eh_ref[...] - mu[None, :]) * inv[None, :] + b_ref[...]
        ef_new = ef_ref[...] + jnp.maximum(bn, 0.0)
        ef_o[...] = ef_new
        t_o[...] = jnp.dot(ef_new, a3_ref[...],
                           preferred_element_type=jnp.float32) + ba_ref[...]

    spec = pl.BlockSpec((_BLK_E, H), lambda i: (i, 0))
    row = pl.BlockSpec((1, H), lambda i: (0, 0))
    return pl.pallas_call(
        body,
        grid=(E // _BLK_E,),
        in_specs=[spec, spec,
                  pl.BlockSpec((NW, 64), lambda i: (0, 0)),
                  row, row,
                  pl.BlockSpec((H, H), lambda i: (0, 0)), row],
        out_specs=[spec, spec],
        out_shape=[jax.ShapeDtypeStruct((E, H), jnp.float32)] * 2,
    )(ef, eh, st, ge_row, be_row, a3, ba_row)


def _node_pre(h, a1, a2, vv, u, bu_row):
    def body(h_ref, a1_ref, a2_ref, v_ref, u_ref, bu_ref,
             p1_ref, p2_ref, pv_ref, hu_ref):
        hb = h_ref[...]
        p1_ref[...] = jnp.dot(hb, a1_ref[...], preferred_element_type=jnp.float32)
        p2_ref[...] = jnp.dot(hb, a2_ref[...], preferred_element_type=jnp.float32)
        pv_ref[...] = jnp.dot(hb, v_ref[...], preferred_element_type=jnp.float32)
        hu_ref[...] = jnp.dot(hb, u_ref[...], preferred_element_type=jnp.float32) + bu_ref[...]

    spec = pl.BlockSpec((_BLK_N, H), lambda i: (i, 0))
    wspec = pl.BlockSpec((H, H), lambda i: (0, 0))
    return pl.pallas_call(
        body,
        grid=(N // _BLK_N,),
        in_specs=[spec, wspec, wspec, wspec, wspec,
                  pl.BlockSpec((1, H), lambda i: (0, 0))],
        out_specs=[spec, spec, spec, spec],
        out_shape=[jax.ShapeDtypeStruct((N, H), jnp.float32)] * 4,
    )(h, a1, a2, vv, u, bu_row)


def _node_hhat(hu, aggden):
    def body(hu_ref, agg_ref, den_ref, hh_ref, st_ref):
        hh = hu_ref[...] + agg_ref[...] / (den_ref[...] + 1e-6)
        hh_ref[...] = hh
        s1 = jnp.sum(hh, axis=0, keepdims=True)
        s2 = jnp.sum(hh * hh, axis=0, keepdims=True)
        blk_stats = jnp.concatenate(
            [s1, s2, jnp.zeros((6, H), jnp.float32)], axis=0)

        @pl.when(pl.program_id(0) == 0)
        def _():
            st_ref[...] = jnp.zeros((8, H), jnp.float32)

        st_ref[...] += blk_stats

    spec = pl.BlockSpec((_BLK_N, H), lambda i: (i, 0))
    return pl.pallas_call(
        body,
        grid=(N // _BLK_N,),
        in_specs=[spec, spec,
                  pl.BlockSpec((_BLK_N, H), lambda i: (i + N // _BLK_N, 0))],
        out_specs=[spec, pl.BlockSpec((8, H), lambda i: (0, 0))],
        out_shape=[
            jax.ShapeDtypeStruct((N, H), jnp.float32),
            jax.ShapeDtypeStruct((8, H), jnp.float32),
        ],
    )(hu, aggden, aggden)


def _node_apply(h, hh, st, gn_row, bnb_row):
    def body(h_ref, hh_ref, st_ref, g_ref, b_ref, o_ref):
        mu = st_ref[0, :] / N
        var = st_ref[1, :] / N - mu * mu
        inv = lax.rsqrt(var + 1e-5)
        bn = g_ref[...] * (hh_ref[...] - mu[None, :]) * inv[None, :] + b_ref[...]
        o_ref[...] = h_ref[...] + jnp.maximum(bn, 0.0)

    spec = pl.BlockSpec((_BLK_N, H), lambda i: (i, 0))
    return pl.pallas_call(
        body,
        grid=(N // _BLK_N,),
        in_specs=[spec, spec,
                  pl.BlockSpec((8, H), lambda i: (0, 0)),
                  pl.BlockSpec((1, H), lambda i: (0, 0)),
                  pl.BlockSpec((1, H), lambda i: (0, 0))],
        out_specs=spec,
        out_shape=jax.ShapeDtypeStruct((N, H), jnp.float32),
    )(h, hh, st, gn_row, bnb_row)


def _node_final(h, w1a, w1b):
    def body(h_ref, wa_ref, wb_ref, qs_ref, qd_ref):
        hb = h_ref[...]
        qs_ref[...] = jnp.dot(hb, wa_ref[...], preferred_element_type=jnp.float32)
        qd_ref[...] = jnp.dot(hb, wb_ref[...], preferred_element_type=jnp.float32)

    spec = pl.BlockSpec((_BLK_N, H), lambda i: (i, 0))
    wspec = pl.BlockSpec((H, H), lambda i: (0, 0))
    return pl.pallas_call(
        body,
        grid=(N // _BLK_N,),
        in_specs=[spec, wspec, wspec],
        out_specs=[spec, spec],
        out_shape=[jax.ShapeDtypeStruct((N, H), jnp.float32)] * 2,
    )(h, w1a, w1b)


def _final(gs, gd, ef, eh, st, ge_row, be_row, w1c, b1_row, w2, b2_row):
    def body(gs_ref, gd_ref, ef_ref, eh_ref, st_ref, g_ref, b_ref,
             wc_ref, b1_ref, w2_ref, b2_ref, o_ref):
        mu, inv = _stats_mean_var(st_ref, float(E))
        bn = g_ref[...] * (eh_ref[...] - mu[None, :]) * inv[None, :] + b_ref[...]
        ef_l = ef_ref[...] + jnp.maximum(bn, 0.0)
        t = jnp.dot(ef_l, wc_ref[...], preferred_element_type=jnp.float32)
        z1 = jnp.maximum(gs_ref[...] + gd_ref[...] + t + b1_ref[...], 0.0)
        o_ref[...] = jnp.dot(z1, w2_ref[...], preferred_element_type=jnp.float32) + b2_ref[...]

    spec = pl.BlockSpec((_BLK_E, H), lambda i: (i, 0))
    row = pl.BlockSpec((1, H), lambda i: (0, 0))
    return pl.pallas_call(
        body,
        grid=(E // _BLK_E,),
        in_specs=[spec, spec, spec, spec,
                  pl.BlockSpec((NW, 64), lambda i: (0, 0)),
                  row, row,
                  pl.BlockSpec((H, H), lambda i: (0, 0)), row,
                  pl.BlockSpec((H, 1), lambda i: (0, 0)),
                  pl.BlockSpec((1, 1), lambda i: (0, 0))],
        out_specs=pl.BlockSpec((_BLK_E, 1), lambda i: (i, 0)),
        out_shape=jax.ShapeDtypeStruct((E, 1), jnp.float32),
    )(gs, gd, ef, eh, st, ge_row, be_row, w1c, b1_row, w2, b2_row)


def kernel(x, e, edge_index, W_pe, b_pe, W_e, b_e, A1, A2, A3, U, V,
           bA, bU, gn, bnb, ge, be, W1, b1, W2, b2):
    src2d = edge_index[0].reshape(NCHUNKS, CH)
    dst2d = edge_index[1].reshape(NCHUNKS, CH)
    zeros = jnp.zeros((ROWS_PER_TILE, H), jnp.float32)

    h = _bcast_init(x, W_pe, b_pe.reshape(1, H), N, _BLK_N)

    ef = None
    eh = None
    st = None
    for l in range(L):
        if l == 0:
            ef, t = _eft0(e, W_e, b_e.reshape(1, H), A3[0], bA[0].reshape(1, H))
        else:
            ef, t = _eft(ef, eh, st, ge[l - 1].reshape(1, H),
                         be[l - 1].reshape(1, H), A3[l], bA[l].reshape(1, H))
        p1, p2, pv, hu = _node_pre(h, A1[l], A2[l], V[l], U[l],
                                   bU[l].reshape(1, H))
        eh, sg, msg, st = _sc_edge(p1, p2, pv, t, dst2d, src2d)
        aggden = _sc_scatter2(msg, sg, dst2d, zeros)
        hh, hst = _node_hhat(hu, aggden)
        h = _node_apply(h, hh, hst, gn[l].reshape(1, H), bnb[l].reshape(1, H))

    qs, qd = _node_final(h, W1[:H], W1[H:2 * H])
    gs, gd = _sc_gather2(qs, qd, dst2d, src2d)
    return _final(gs, gd, ef, eh, st, ge[L - 1].reshape(1, H),
                  be[L - 1].reshape(1, H), W1[2 * H:], b1.reshape(1, H), W2,
                  b2.reshape(1, 1))


# speedup vs baseline: 3.3068x; 1.0457x over previous
import functools

import jax
import jax.numpy as jnp
from jax import lax
from jax.experimental import pallas as pl
from jax.experimental.pallas import tpu as pltpu
from jax.experimental.pallas import tpu_sc as plsc

N = 50000
E = 800000
H = 32
L = 4

NCORES = 2
NSUB = 16
NW = NCORES * NSUB
CH = 128
NCHUNKS = E // CH
KSG = 2
SG = KSG * CH
NSG = E // SG
ROWS_PER_TILE = N // NSUB

_mesh = plsc.VectorSubcoreMesh(
    core_axis_name="c", subcore_axis_name="s", num_cores=NCORES,
    num_subcores=NSUB)

_SC_PARAMS = pltpu.CompilerParams(use_tc_tiling_on_sc=False)


@functools.partial(
    pl.kernel,
    out_type=(
        jax.ShapeDtypeStruct((E, H), jnp.float32),
        jax.ShapeDtypeStruct((E, H), jnp.float32),
        jax.ShapeDtypeStruct((E, H), jnp.float32),
        jax.ShapeDtypeStruct((NW, 64), jnp.float32),
    ),
    mesh=_mesh,
    compiler_params=_SC_PARAMS,
    scratch_types=[
        pltpu.VMEM((KSG, CH), jnp.int32),
        pltpu.VMEM((KSG, CH), jnp.int32),
        pltpu.VMEM((KSG, CH), jnp.int32),
        pltpu.VMEM((KSG, CH), jnp.int32),
        pltpu.VMEM((SG, H), jnp.float32),
        pltpu.VMEM((SG, H), jnp.float32),
        pltpu.VMEM((SG, H), jnp.float32),
        pltpu.VMEM((SG, H), jnp.float32),
        pltpu.VMEM((SG, H), jnp.float32),
        pltpu.VMEM((SG, H), jnp.float32),
        pltpu.VMEM((SG, H), jnp.float32),
        pltpu.VMEM((SG, H), jnp.float32),
        pltpu.VMEM((64,), jnp.float32),
        pltpu.SemaphoreType.DMA,
        pltpu.SemaphoreType.DMA,
    ],
)
def _sc_edge(p1, p2, pv, t, dst2d, src2d, ehat, sig, msg, stats,
             idxda, idxsa, idxdb, idxsb, g1a, g2a, gva, ta,
             g1b, g2b, gvb, tb, sbuf, sem_g, sem_w):
    c = lax.axis_index("c")
    s = lax.axis_index("s")
    w = s * NCORES + c
    nbase = NSG // NW
    rem = NSG % NW
    sg0 = w * nbase + jnp.minimum(w, rem)
    nsg = nbase + jnp.where(w < rem, 1, 0)
    npairs = nsg // 2

    def start_group(g, idxd, idxs, g1, g2, gv, tg):
        sgi = sg0 + g
        crow = sgi * KSG
        row0 = sgi * SG
        pltpu.sync_copy(dst2d.at[pl.ds(crow, KSG)], idxd)
        pltpu.sync_copy(src2d.at[pl.ds(crow, KSG)], idxs)
        pltpu.async_copy(t.at[pl.ds(row0, SG)], tg, sem_g)
        for k in range(KSG):
            pltpu.async_copy(p1.at[idxd.at[k]], g1.at[pl.ds(k * CH, CH)], sem_g)
            pltpu.async_copy(p2.at[idxs.at[k]], g2.at[pl.ds(k * CH, CH)], sem_g)
            pltpu.async_copy(pv.at[idxs.at[k]], gv.at[pl.ds(k * CH, CH)], sem_g)

    def wait_gathers(g1, g2, gv, tg):
        for buf in (g1, g2, gv, tg):
            pltpu.make_async_copy(t.at[pl.ds(0, SG)], buf, sem_g).wait()

    def wait_writes():
        for _ in range(3):
            pltpu.make_async_copy(t.at[pl.ds(0, SG)], g1a, sem_w).wait()

    def compute(g1, g2, gv, tg, st):
        def row(r, st2):
            a0, a1, q0, q1 = st2
            e0 = g1[r, pl.ds(0, 16)] + g2[r, pl.ds(0, 16)] + tg[r, pl.ds(0, 16)]
            e1 = g1[r, pl.ds(16, 16)] + g2[r, pl.ds(16, 16)] + tg[r, pl.ds(16, 16)]
            s0 = 1.0 / (1.0 + jnp.exp(-e0))
            s1 = 1.0 / (1.0 + jnp.exp(-e1))
            g1[r, pl.ds(0, 16)] = e0
            g1[r, pl.ds(16, 16)] = e1
            g2[r, pl.ds(0, 16)] = s0
            g2[r, pl.ds(16, 16)] = s1
            gv[r, pl.ds(0, 16)] = s0 * gv[r, pl.ds(0, 16)]
            gv[r, pl.ds(16, 16)] = s1 * gv[r, pl.ds(16, 16)]
            return (a0 + e0, a1 + e1, q0 + e0 * e0, q1 + e1 * e1)

        return lax.fori_loop(0, SG, row, st)

    def issue_writes(g, g1, g2, gv):
        row0 = (sg0 + g) * SG
        pltpu.async_copy(g1, ehat.at[pl.ds(row0, SG)], sem_w)
        pltpu.async_copy(g2, sig.at[pl.ds(row0, SG)], sem_w)
        pltpu.async_copy(gv, msg.at[pl.ds(row0, SG)], sem_w)

    z = jnp.zeros((16,), jnp.float32)
    start_group(0, idxda, idxsa, g1a, g2a, gva, ta)

    def body(i, st):
        wait_gathers(g1a, g2a, gva, ta)

        @pl.when(i > 0)
        def _():
            wait_writes()

        start_group(2 * i + 1, idxdb, idxsb, g1b, g2b, gvb, tb)
        st = compute(g1a, g2a, gva, ta, st)
        issue_writes(2 * i, g1a, g2a, gva)

        wait_gathers(g1b, g2b, gvb, tb)
        wait_writes()

        @pl.when(2 * i + 2 < nsg)
        def _():
            start_group(2 * i + 2, idxda, idxsa, g1a, g2a, gva, ta)

        st = compute(g1b, g2b, gvb, tb, st)
        issue_writes(2 * i + 1, g1b, g2b, gvb)
        return st

    st = lax.fori_loop(0, npairs, body, (z, z, z, z))
    wait_writes()

    def tail_body(j, st2):
        wait_gathers(g1a, g2a, gva, ta)
        st2 = compute(g1a, g2a, gva, ta, st2)
        row0 = (sg0 + 2 * npairs) * SG
        pltpu.sync_copy(g1a, ehat.at[pl.ds(row0, SG)])
        pltpu.sync_copy(g2a, sig.at[pl.ds(row0, SG)])
        pltpu.sync_copy(gva, msg.at[pl.ds(row0, SG)])
        return st2

    a0, a1, q0, q1 = lax.fori_loop(0, nsg - 2 * npairs, tail_body, st)
    sbuf[pl.ds(0, 16)] = a0
    sbuf[pl.ds(16, 16)] = a1
    sbuf[pl.ds(32, 16)] = q0
    sbuf[pl.ds(48, 16)] = q1
    pltpu.sync_copy(sbuf, stats.at[w])


GRP = 3
NGRP = (NCHUNKS // NSUB) // GRP


@functools.partial(
    pl.kernel,
    out_type=jax.ShapeDtypeStruct((2 * N, H), jnp.float32),
    mesh=_mesh,
    compiler_params=_SC_PARAMS,
    scratch_types=[
        pltpu.VMEM_SHARED((N, H), jnp.float32),
        pltpu.VMEM((GRP, CH), jnp.int32),
        pltpu.VMEM((GRP, CH), jnp.int32),
        pltpu.VMEM((GRP * CH, H), jnp.float32),
        pltpu.VMEM((GRP * CH, H), jnp.float32),
        pltpu.SemaphoreType.DMA,
        pltpu.SemaphoreType.DMA,
    ],
)
def _sc_scatter2(msg, sig, dst2d, zeros, out, acc, idx0, idx1, val0, val1,
                 sem_l, sem_s):
    c = lax.axis_index("c")
    s = lax.axis_index("s")
    pltpu.sync_copy(zeros.at[pl.ds(0, ROWS_PER_TILE)],
                    acc.at[pl.ds(s * ROWS_PER_TILE, ROWS_PER_TILE)])
    plsc.subcore_barrier()

    nbase = NCHUNKS // NSUB
    rem = NCHUNKS % NSUB
    start = s * nbase + jnp.minimum(s, rem)
    cnt = nbase + jnp.where(s < rem, 1, 0)

    def pipe(vhbm):
        def load(g, idxs, vals):
            ch0 = start + g * GRP
            pltpu.async_copy(dst2d.at[pl.ds(ch0, GRP)], idxs, sem_l)
            pltpu.async_copy(vhbm.at[pl.ds(ch0 * CH, GRP * CH)], vals, sem_l)

        def wait_load(idxs, vals):
            pltpu.make_async_copy(dst2d.at[pl.ds(0, GRP)], idxs, sem_l).wait()
            pltpu.make_async_copy(vhbm.at[pl.ds(0, GRP * CH)], vals,
                                  sem_l).wait()

        def scat(idxs, vals):
            for k in range(GRP):
                pltpu.async_copy(vals.at[pl.ds(k * CH, CH)],
                                 acc.at[idxs.at[k]], sem_s, add=True)

        def wait_scat(vals):
            pltpu.make_async_copy(vhbm.at[pl.ds(0, GRP * CH)], vals,
                                  sem_s).wait()

        load(0, idx0, val0)

        def body(i, carry):
            ga = 2 * i
            wait_load(idx0, val0)

            @pl.when(i > 0)
            def _():
                wait_scat(val1)

            scat(idx0, val0)
            load(ga + 1, idx1, val1)
            wait_load(idx1, val1)
            wait_scat(val0)
            scat(idx1, val1)

            @pl.when(i < NGRP // 2 - 1)
            def _():
                load(ga + 2, idx0, val0)

            return carry

        lax.fori_loop(0, NGRP // 2, body, 0)
        wait_scat(val1)

        def tbody(j, carry):
            ch = start + NGRP * GRP + j
            pltpu.sync_copy(dst2d.at[ch], idx0.at[0])
            pltpu.sync_copy(vhbm.at[pl.ds(ch * CH, CH)],
                            val0.at[pl.ds(0, CH)])
            pltpu.sync_copy(val0.at[pl.ds(0, CH)], acc.at[idx0.at[0]],
                            add=True)
            return carry

        lax.fori_loop(0, cnt - NGRP * GRP, tbody, 0)

    @pl.when(c == 0)
    def _():
        pipe(msg)

    @pl.when(c == 1)
    def _():
        pipe(sig)

    plsc.subcore_barrier()
    pltpu.sync_copy(acc.at[pl.ds(s * ROWS_PER_TILE, ROWS_PER_TILE)],
                    out.at[pl.ds(c * N + s * ROWS_PER_TILE, ROWS_PER_TILE)])


@functools.partial(
    pl.kernel,
    out_type=(
        jax.ShapeDtypeStruct((E, H), jnp.float32),
        jax.ShapeDtypeStruct((E, H), jnp.float32),
    ),
    mesh=_mesh,
    compiler_params=_SC_PARAMS,
    scratch_types=[
        pltpu.VMEM((KSG, CH), jnp.int32),
        pltpu.VMEM((KSG, CH), jnp.int32),
        pltpu.VMEM((SG, H), jnp.float32),
        pltpu.VMEM((SG, H), jnp.float32),
        pltpu.SemaphoreType.DMA,
    ],
)
def _sc_gather2(qs, qd, dst2d, src2d, gs, gd, idxd, idxs, b1, b2, sem):
    c = lax.axis_index("c")
    s = lax.axis_index("s")
    w = s * NCORES + c
    nbase = NSG // NW
    rem = NSG % NW
    sg0 = w * nbase + jnp.minimum(w, rem)
    nsg = nbase + jnp.where(w < rem, 1, 0)

    def body(i, carry):
        sgi = sg0 + i
        crow = sgi * KSG
        row0 = sgi * SG
        pltpu.sync_copy(dst2d.at[pl.ds(crow, KSG)], idxd)
        pltpu.sync_copy(src2d.at[pl.ds(crow, KSG)], idxs)
        cps = []
        for k in range(KSG):
            cps.append(pltpu.async_copy(
                qs.at[idxs.at[k]], b1.at[pl.ds(k * CH, CH)], sem))
            cps.append(pltpu.async_copy(
                qd.at[idxd.at[k]], b2.at[pl.ds(k * CH, CH)], sem))
        for cp in cps:
            cp.wait()
        c1 = pltpu.async_copy(b1, gs.at[pl.ds(row0, SG)], sem)
        c2 = pltpu.async_copy(b2, gd.at[pl.ds(row0, SG)], sem)
        c1.wait()
        c2.wait()
        return carry

    lax.fori_loop(0, nsg, body, 0)


_BLK_E = 4000
_BLK_N = 5000


def _stats_mean_var(st_ref, count):
    ssum = jnp.sum(st_ref[...], axis=0)
    mu = jnp.concatenate([ssum[0:16], ssum[16:32]]) / count
    msq = jnp.concatenate([ssum[32:48], ssum[48:64]]) / count
    var = msq - mu * mu
    return mu, lax.rsqrt(var + 1e-5)


def _bcast_init(v, w_row, b_row, rows, blk):
    def body(v_ref, w_ref, b_ref, o_ref):
        o_ref[...] = v_ref[...] * w_ref[...] + b_ref[...]

    return pl.pallas_call(
        body,
        grid=(rows // blk,),
        in_specs=[
            pl.BlockSpec((blk, 1), lambda i: (i, 0)),
            pl.BlockSpec((1, H), lambda i: (0, 0)),
            pl.BlockSpec((1, H), lambda i: (0, 0)),
        ],
        out_specs=pl.BlockSpec((blk, H), lambda i: (i, 0)),
        out_shape=jax.ShapeDtypeStruct((rows, H), jnp.float32),
    )(v, w_row, b_row)


def _eft0(e, w_row, b_row, a3, ba_row):
    def body(e_ref, w_ref, b_ref, a3_ref, ba_ref, ef_ref, t_ref):
        ef = e_ref[...] * w_ref[...] + b_ref[...]
        ef_ref[...] = ef
        t_ref[...] = jnp.dot(ef, a3_ref[...],
                             preferred_element_type=jnp.float32) + ba_ref[...]

    spec = pl.BlockSpec((_BLK_E, H), lambda i: (i, 0))
    return pl.pallas_call(
        body,
        grid=(E // _BLK_E,),
        in_specs=[pl.BlockSpec((_BLK_E, 1), lambda i: (i, 0)),
                  pl.BlockSpec((1, H), lambda i: (0, 0)),
                  pl.BlockSpec((1, H), lambda i: (0, 0)),
                  pl.BlockSpec((H, H), lambda i: (0, 0)),
                  pl.BlockSpec((1, H), lambda i: (0, 0))],
        out_specs=[spec, spec],
        out_shape=[jax.ShapeDtypeStruct((E, H), jnp.float32)] * 2,
    )(e, w_row, b_row, a3, ba_row)


def _eft(ef, eh, st, ge_row, be_row, a3, ba_row):
    def body(ef_ref, eh_ref, st_ref, g_ref, b_ref, a3_ref, ba_ref,
             ef_o, t_o):
        mu, inv = _stats_mean_var(st_ref, float(E))
        bn = g_ref[...] * (eh_ref[...] - mu[None, :]) * inv[None, :] + b_ref[...]
        ef_new = ef_ref[...] + jnp.maximum(bn, 0.0)
        ef_o[...] = ef_new
        t_o[...] = jnp.dot(ef_new, a3_ref[...],
                           preferred_element_type=jnp.float32) + ba_ref[...]

    spec = pl.BlockSpec((_BLK_E, H), lambda i: (i, 0))
    row = pl.BlockSpec((1, H), lambda i: (0, 0))
    return pl.pallas_call(
        body,
        grid=(E // _BLK_E,),
        in_specs=[spec, spec,
                  pl.BlockSpec((NW, 64), lambda i: (0, 0)),
                  row, row,
                  pl.BlockSpec((H, H), lambda i: (0, 0)), row],
        out_specs=[spec, spec],
        out_shape=[jax.ShapeDtypeStruct((E, H), jnp.float32)] * 2,
    )(ef, eh, st, ge_row, be_row, a3, ba_row)


def _node_pre(h, a1, a2, vv, u, bu_row):
    def body(h_ref, a1_ref, a2_ref, v_ref, u_ref, bu_ref,
             p1_ref, p2_ref, pv_ref, hu_ref):
        hb = h_ref[...]
        p1_ref[...] = jnp.dot(hb, a1_ref[...], preferred_element_type=jnp.float32)
        p2_ref[...] = jnp.dot(hb, a2_ref[...], preferred_element_type=jnp.float32)
        pv_ref[...] = jnp.dot(hb, v_ref[...], preferred_element_type=jnp.float32)
        hu_ref[...] = jnp.dot(hb, u_ref[...], preferred_element_type=jnp.float32) + bu_ref[...]

    spec = pl.BlockSpec((_BLK_N, H), lambda i: (i, 0))
    wspec = pl.BlockSpec((H, H), lambda i: (0, 0))
    return pl.pallas_call(
        body,
        grid=(N // _BLK_N,),
        in_specs=[spec, wspec, wspec, wspec, wspec,
                  pl.BlockSpec((1, H), lambda i: (0, 0))],
        out_specs=[spec, spec, spec, spec],
        out_shape=[jax.ShapeDtypeStruct((N, H), jnp.float32)] * 4,
    )(h, a1, a2, vv, u, bu_row)


def _node_hhat(hu, aggden):
    def body(hu_ref, agg_ref, den_ref, hh_ref, st_ref):
        hh = hu_ref[...] + agg_ref[...] / (den_ref[...] + 1e-6)
        hh_ref[...] = hh
        s1 = jnp.sum(hh, axis=0, keepdims=True)
        s2 = jnp.sum(hh * hh, axis=0, keepdims=True)
        blk_stats = jnp.concatenate(
            [s1, s2, jnp.zeros((6, H), jnp.float32)], axis=0)

        @pl.when(pl.program_id(0) == 0)
        def _():
            st_ref[...] = jnp.zeros((8, H), jnp.float32)

        st_ref[...] += blk_stats

    spec = pl.BlockSpec((_BLK_N, H), lambda i: (i, 0))
    return pl.pallas_call(
        body,
        grid=(N // _BLK_N,),
        in_specs=[spec, spec,
                  pl.BlockSpec((_BLK_N, H), lambda i: (i + N // _BLK_N, 0))],
        out_specs=[spec, pl.BlockSpec((8, H), lambda i: (0, 0))],
        out_shape=[
            jax.ShapeDtypeStruct((N, H), jnp.float32),
            jax.ShapeDtypeStruct((8, H), jnp.float32),
        ],
    )(hu, aggden, aggden)


def _node_apply(h, hh, st, gn_row, bnb_row):
    def body(h_ref, hh_ref, st_ref, g_ref, b_ref, o_ref):
        mu = st_ref[0, :] / N
        var = st_ref[1, :] / N - mu * mu
        inv = lax.rsqrt(var + 1e-5)
        bn = g_ref[...] * (hh_ref[...] - mu[None, :]) * inv[None, :] + b_ref[...]
        o_ref[...] = h_ref[...] + jnp.maximum(bn, 0.0)

    spec = pl.BlockSpec((_BLK_N, H), lambda i: (i, 0))
    return pl.pallas_call(
        body,
        grid=(N // _BLK_N,),
        in_specs=[spec, spec,
                  pl.BlockSpec((8, H), lambda i: (0, 0)),
                  pl.BlockSpec((1, H), lambda i: (0, 0)),
                  pl.BlockSpec((1, H), lambda i: (0, 0))],
        out_specs=spec,
        out_shape=jax.ShapeDtypeStruct((N, H), jnp.float32),
    )(h, hh, st, gn_row, bnb_row)


def _node_final(h, w1a, w1b):
    def body(h_ref, wa_ref, wb_ref, qs_ref, qd_ref):
        hb = h_ref[...]
        qs_ref[...] = jnp.dot(hb, wa_ref[...], preferred_element_type=jnp.float32)
        qd_ref[...] = jnp.dot(hb, wb_ref[...], preferred_element_type=jnp.float32)

    spec = pl.BlockSpec((_BLK_N, H), lambda i: (i, 0))
    wspec = pl.BlockSpec((H, H), lambda i: (0, 0))
    return pl.pallas_call(
        body,
        grid=(N // _BLK_N,),
        in_specs=[spec, wspec, wspec],
        out_specs=[spec, spec],
        out_shape=[jax.ShapeDtypeStruct((N, H), jnp.float32)] * 2,
    )(h, w1a, w1b)


def _final(gs, gd, ef, eh, st, ge_row, be_row, w1c, b1_row, w2, b2_row):
    def body(gs_ref, gd_ref, ef_ref, eh_ref, st_ref, g_ref, b_ref,
             wc_ref, b1_ref, w2_ref, b2_ref, o_ref):
        mu, inv = _stats_mean_var(st_ref, float(E))
        bn = g_ref[...] * (eh_ref[...] - mu[None, :]) * inv[None, :] + b_ref[...]
        ef_l = ef_ref[...] + jnp.maximum(bn, 0.0)
        t = jnp.dot(ef_l, wc_ref[...], preferred_element_type=jnp.float32)
        z1 = jnp.maximum(gs_ref[...] + gd_ref[...] + t + b1_ref[...], 0.0)
        o_ref[...] = jnp.dot(z1, w2_ref[...], preferred_element_type=jnp.float32) + b2_ref[...]

    spec = pl.BlockSpec((_BLK_E, H), lambda i: (i, 0))
    row = pl.BlockSpec((1, H), lambda i: (0, 0))
    return pl.pallas_call(
        body,
        grid=(E // _BLK_E,),
        in_specs=[spec, spec, spec, spec,
                  pl.BlockSpec((NW, 64), lambda i: (0, 0)),
                  row, row,
                  pl.BlockSpec((H, H), lambda i: (0, 0)), row,
                  pl.BlockSpec((H, 1), lambda i: (0, 0)),
                  pl.BlockSpec((1, 1), lambda i: (0, 0))],
        out_specs=pl.BlockSpec((_BLK_E, 1), lambda i: (i, 0)),
        out_shape=jax.ShapeDtypeStruct((E, 1), jnp.float32),
    )(gs, gd, ef, eh, st, ge_row, be_row, w1c, b1_row, w2, b2_row)


def kernel(x, e, edge_index, W_pe, b_pe, W_e, b_e, A1, A2, A3, U, V,
           bA, bU, gn, bnb, ge, be, W1, b1, W2, b2):
    src2d = edge_index[0].reshape(NCHUNKS, CH)
    dst2d = edge_index[1].reshape(NCHUNKS, CH)
    zeros = jnp.zeros((ROWS_PER_TILE, H), jnp.float32)

    h = _bcast_init(x, W_pe, b_pe.reshape(1, H), N, _BLK_N)

    ef = None
    eh = None
    st = None
    for l in range(L):
        if l == 0:
            ef, t = _eft0(e, W_e, b_e.reshape(1, H), A3[0], bA[0].reshape(1, H))
        else:
            ef, t = _eft(ef, eh, st, ge[l - 1].reshape(1, H),
                         be[l - 1].reshape(1, H), A3[l], bA[l].reshape(1, H))
        p1, p2, pv, hu = _node_pre(h, A1[l], A2[l], V[l], U[l],
                                   bU[l].reshape(1, H))
        eh, sg, msg, st = _sc_edge(p1, p2, pv, t, dst2d, src2d)
        aggden = _sc_scatter2(msg, sg, dst2d, zeros)
        hh, hst = _node_hhat(hu, aggden)
        h = _node_apply(h, hh, hst, gn[l].reshape(1, H), bnb[l].reshape(1, H))

    qs, qd = _node_final(h, W1[:H], W1[H:2 * H])
    gs, gd = _sc_gather2(qs, qd, dst2d, src2d)
    return _final(gs, gd, ef, eh, st, ge[L - 1].reshape(1, H),
                  be[L - 1].reshape(1, H), W1[2 * H:], b1.reshape(1, H), W2,
                  b2.reshape(1, 1))


# speedup vs baseline: 5.6490x; 1.7083x over previous
import functools

import jax
import jax.numpy as jnp
from jax import lax
from jax.experimental import pallas as pl
from jax.experimental.pallas import tpu as pltpu
from jax.experimental.pallas import tpu_sc as plsc

N = 50000
E = 800000
H = 32
L = 4

NCORES = 2
NSUB = 16
NW = NCORES * NSUB
CH = 128
NCHUNKS = E // CH
KSG = 2
SG = KSG * CH
NSG = E // SG
ROWS_PER_TILE = N // NSUB

_mesh = plsc.VectorSubcoreMesh(
    core_axis_name="c", subcore_axis_name="s", num_cores=NCORES,
    num_subcores=NSUB)

_SC_PARAMS = pltpu.CompilerParams(use_tc_tiling_on_sc=False)


@functools.partial(
    pl.kernel,
    out_type=(
        jax.ShapeDtypeStruct((E, H), jnp.float32),
        jax.ShapeDtypeStruct((E, H), jnp.float32),
        jax.ShapeDtypeStruct((E, H), jnp.float32),
        jax.ShapeDtypeStruct((NW, 64), jnp.float32),
    ),
    mesh=_mesh,
    compiler_params=_SC_PARAMS,
    scratch_types=[
        pltpu.VMEM((KSG, CH), jnp.int32),
        pltpu.VMEM((KSG, CH), jnp.int32),
        pltpu.VMEM((KSG, CH), jnp.int32),
        pltpu.VMEM((KSG, CH), jnp.int32),
        pltpu.VMEM((SG, H), jnp.float32),
        pltpu.VMEM((SG, H), jnp.float32),
        pltpu.VMEM((SG, H), jnp.float32),
        pltpu.VMEM((SG, H), jnp.float32),
        pltpu.VMEM((SG, H), jnp.float32),
        pltpu.VMEM((SG, H), jnp.float32),
        pltpu.VMEM((SG, H), jnp.float32),
        pltpu.VMEM((SG, H), jnp.float32),
        pltpu.VMEM((64,), jnp.float32),
        pltpu.SemaphoreType.DMA,
        pltpu.SemaphoreType.DMA,
    ],
)
def _sc_edge(p1, p2, pv, t, dst2d, src2d, ehat, sig, msg, stats,
             idxda, idxsa, idxdb, idxsb, g1a, g2a, gva, ta,
             g1b, g2b, gvb, tb, sbuf, sem_g, sem_w):
    c = lax.axis_index("c")
    s = lax.axis_index("s")
    w = s * NCORES + c
    nbase = NSG // NW
    rem = NSG % NW
    sg0 = w * nbase + jnp.minimum(w, rem)
    nsg = nbase + jnp.where(w < rem, 1, 0)
    npairs = nsg // 2

    def start_group(g, idxd, idxs, g1, g2, gv, tg):
        sgi = sg0 + g
        crow = sgi * KSG
        row0 = sgi * SG
        pltpu.sync_copy(dst2d.at[pl.ds(crow, KSG)], idxd)
        pltpu.sync_copy(src2d.at[pl.ds(crow, KSG)], idxs)
        pltpu.async_copy(t.at[pl.ds(row0, SG)], tg, sem_g)
        for k in range(KSG):
            pltpu.async_copy(p1.at[idxd.at[k]], g1.at[pl.ds(k * CH, CH)], sem_g)
            pltpu.async_copy(p2.at[idxs.at[k]], g2.at[pl.ds(k * CH, CH)], sem_g)
            pltpu.async_copy(pv.at[idxs.at[k]], gv.at[pl.ds(k * CH, CH)], sem_g)

    def wait_gathers(g1, g2, gv, tg):
        for buf in (g1, g2, gv, tg):
            pltpu.make_async_copy(t.at[pl.ds(0, SG)], buf, sem_g).wait()

    def wait_writes():
        for _ in range(3):
            pltpu.make_async_copy(t.at[pl.ds(0, SG)], g1a, sem_w).wait()

    def compute(g1, g2, gv, tg, st):
        def row(r, st2):
            a0, a1, q0, q1 = st2
            e0 = g1[r, pl.ds(0, 16)] + g2[r, pl.ds(0, 16)] + tg[r, pl.ds(0, 16)]
            e1 = g1[r, pl.ds(16, 16)] + g2[r, pl.ds(16, 16)] + tg[r, pl.ds(16, 16)]
            s0 = 1.0 / (1.0 + jnp.exp(-e0))
            s1 = 1.0 / (1.0 + jnp.exp(-e1))
            g1[r, pl.ds(0, 16)] = e0
            g1[r, pl.ds(16, 16)] = e1
            g2[r, pl.ds(0, 16)] = s0
            g2[r, pl.ds(16, 16)] = s1
            gv[r, pl.ds(0, 16)] = s0 * gv[r, pl.ds(0, 16)]
            gv[r, pl.ds(16, 16)] = s1 * gv[r, pl.ds(16, 16)]
            return (a0 + e0, a1 + e1, q0 + e0 * e0, q1 + e1 * e1)

        return lax.fori_loop(0, SG, row, st)

    def issue_writes(g, g1, g2, gv):
        row0 = (sg0 + g) * SG
        pltpu.async_copy(g1, ehat.at[pl.ds(row0, SG)], sem_w)
        pltpu.async_copy(g2, sig.at[pl.ds(row0, SG)], sem_w)
        pltpu.async_copy(gv, msg.at[pl.ds(row0, SG)], sem_w)

    z = jnp.zeros((16,), jnp.float32)
    start_group(0, idxda, idxsa, g1a, g2a, gva, ta)

    def body(i, st):
        wait_gathers(g1a, g2a, gva, ta)

        @pl.when(i > 0)
        def _():
            wait_writes()

        start_group(2 * i + 1, idxdb, idxsb, g1b, g2b, gvb, tb)
        st = compute(g1a, g2a, gva, ta, st)
        issue_writes(2 * i, g1a, g2a, gva)

        wait_gathers(g1b, g2b, gvb, tb)
        wait_writes()

        @pl.when(2 * i + 2 < nsg)
        def _():
            start_group(2 * i + 2, idxda, idxsa, g1a, g2a, gva, ta)

        st = compute(g1b, g2b, gvb, tb, st)
        issue_writes(2 * i + 1, g1b, g2b, gvb)
        return st

    st = lax.fori_loop(0, npairs, body, (z, z, z, z))
    wait_writes()

    def tail_body(j, st2):
        wait_gathers(g1a, g2a, gva, ta)
        st2 = compute(g1a, g2a, gva, ta, st2)
        row0 = (sg0 + 2 * npairs) * SG
        pltpu.sync_copy(g1a, ehat.at[pl.ds(row0, SG)])
        pltpu.sync_copy(g2a, sig.at[pl.ds(row0, SG)])
        pltpu.sync_copy(gva, msg.at[pl.ds(row0, SG)])
        return st2

    a0, a1, q0, q1 = lax.fori_loop(0, nsg - 2 * npairs, tail_body, st)
    sbuf[pl.ds(0, 16)] = a0
    sbuf[pl.ds(16, 16)] = a1
    sbuf[pl.ds(32, 16)] = q0
    sbuf[pl.ds(48, 16)] = q1
    pltpu.sync_copy(sbuf, stats.at[w])


GRP = 3
NGRP = (NCHUNKS // NSUB) // GRP


@functools.partial(
    pl.kernel,
    out_type=jax.ShapeDtypeStruct((2 * N, H), jnp.float32),
    mesh=_mesh,
    compiler_params=_SC_PARAMS,
    scratch_types=[
        pltpu.VMEM_SHARED((N, H), jnp.float32),
        pltpu.VMEM((GRP, CH), jnp.int32),
        pltpu.VMEM((GRP, CH), jnp.int32),
        pltpu.VMEM((GRP * CH, H), jnp.float32),
        pltpu.VMEM((GRP * CH, H), jnp.float32),
        pltpu.SemaphoreType.DMA,
        pltpu.SemaphoreType.DMA,
    ],
)
def _sc_scatter2(msg, sig, dst2d, zeros, out, acc, idx0, idx1, val0, val1,
                 sem_l, sem_s):
    c = lax.axis_index("c")
    s = lax.axis_index("s")
    pltpu.sync_copy(zeros.at[pl.ds(0, ROWS_PER_TILE)],
                    acc.at[pl.ds(s * ROWS_PER_TILE, ROWS_PER_TILE)])
    plsc.subcore_barrier()

    nbase = NCHUNKS // NSUB
    rem = NCHUNKS % NSUB
    start = s * nbase + jnp.minimum(s, rem)
    cnt = nbase + jnp.where(s < rem, 1, 0)

    def pipe(vhbm):
        def load(g, idxs, vals):
            ch0 = start + g * GRP
            pltpu.async_copy(dst2d.at[pl.ds(ch0, GRP)], idxs, sem_l)
            pltpu.async_copy(vhbm.at[pl.ds(ch0 * CH, GRP * CH)], vals, sem_l)

        def wait_load(idxs, vals):
            pltpu.make_async_copy(dst2d.at[pl.ds(0, GRP)], idxs, sem_l).wait()
            pltpu.make_async_copy(vhbm.at[pl.ds(0, GRP * CH)], vals,
                                  sem_l).wait()

        def scat(idxs, vals):
            for k in range(GRP):
                pltpu.async_copy(vals.at[pl.ds(k * CH, CH)],
                                 acc.at[idxs.at[k]], sem_s, add=True)

        def wait_scat(vals):
            pltpu.make_async_copy(vhbm.at[pl.ds(0, GRP * CH)], vals,
                                  sem_s).wait()

        load(0, idx0, val0)

        def body(i, carry):
            ga = 2 * i
            wait_load(idx0, val0)

            @pl.when(i > 0)
            def _():
                wait_scat(val1)

            scat(idx0, val0)
            load(ga + 1, idx1, val1)
            wait_load(idx1, val1)
            wait_scat(val0)
            scat(idx1, val1)

            @pl.when(i < NGRP // 2 - 1)
            def _():
                load(ga + 2, idx0, val0)

            return carry

        lax.fori_loop(0, NGRP // 2, body, 0)
        wait_scat(val1)

        def tbody(j, carry):
            ch = start + NGRP * GRP + j
            pltpu.sync_copy(dst2d.at[ch], idx0.at[0])
            pltpu.sync_copy(vhbm.at[pl.ds(ch * CH, CH)],
                            val0.at[pl.ds(0, CH)])
            pltpu.sync_copy(val0.at[pl.ds(0, CH)], acc.at[idx0.at[0]],
                            add=True)
            return carry

        lax.fori_loop(0, cnt - NGRP * GRP, tbody, 0)

    @pl.when(c == 0)
    def _():
        pipe(msg)

    @pl.when(c == 1)
    def _():
        pipe(sig)

    plsc.subcore_barrier()
    pltpu.sync_copy(acc.at[pl.ds(s * ROWS_PER_TILE, ROWS_PER_TILE)],
                    out.at[pl.ds(c * N + s * ROWS_PER_TILE, ROWS_PER_TILE)])


@functools.partial(
    pl.kernel,
    out_type=(
        jax.ShapeDtypeStruct((E, H), jnp.float32),
        jax.ShapeDtypeStruct((E, H), jnp.float32),
    ),
    mesh=_mesh,
    compiler_params=_SC_PARAMS,
    scratch_types=[
        pltpu.VMEM((KSG, CH), jnp.int32),
        pltpu.VMEM((KSG, CH), jnp.int32),
        pltpu.VMEM((SG, H), jnp.float32),
        pltpu.VMEM((SG, H), jnp.float32),
        pltpu.SemaphoreType.DMA,
    ],
)
def _sc_gather2(qs, qd, dst2d, src2d, gs, gd, idxd, idxs, b1, b2, sem):
    c = lax.axis_index("c")
    s = lax.axis_index("s")
    w = s * NCORES + c
    nbase = NSG // NW
    rem = NSG % NW
    sg0 = w * nbase + jnp.minimum(w, rem)
    nsg = nbase + jnp.where(w < rem, 1, 0)

    def body(i, carry):
        sgi = sg0 + i
        crow = sgi * KSG
        row0 = sgi * SG
        pltpu.sync_copy(dst2d.at[pl.ds(crow, KSG)], idxd)
        pltpu.sync_copy(src2d.at[pl.ds(crow, KSG)], idxs)
        cps = []
        for k in range(KSG):
            cps.append(pltpu.async_copy(
                qs.at[idxs.at[k]], b1.at[pl.ds(k * CH, CH)], sem))
            cps.append(pltpu.async_copy(
                qd.at[idxd.at[k]], b2.at[pl.ds(k * CH, CH)], sem))
        for cp in cps:
            cp.wait()
        c1 = pltpu.async_copy(b1, gs.at[pl.ds(row0, SG)], sem)
        c2 = pltpu.async_copy(b2, gd.at[pl.ds(row0, SG)], sem)
        c1.wait()
        c2.wait()
        return carry

    lax.fori_loop(0, nsg, body, 0)


E4 = E // 4
_BLK_E = 4000
_BLK_P = 2000
_BLK_N = 5000


def _stats_mean_var128(st_ref, count):
    ssum = jnp.sum(st_ref[...], axis=0)
    mu32 = jnp.concatenate([ssum[0:16], ssum[16:32]])
    msq32 = jnp.concatenate([ssum[32:48], ssum[48:64]])
    mu = jnp.tile(mu32, 4) / count
    msq = jnp.tile(msq32, 4) / count
    var = msq - mu * mu
    return mu, lax.rsqrt(var + 1e-5)


def _bcast_init(v, w_row, b_row, rows, blk):
    def body(v_ref, w_ref, b_ref, o_ref):
        o_ref[...] = v_ref[...] * w_ref[...] + b_ref[...]

    return pl.pallas_call(
        body,
        grid=(rows // blk,),
        in_specs=[
            pl.BlockSpec((blk, 1), lambda i: (i, 0)),
            pl.BlockSpec((1, H), lambda i: (0, 0)),
            pl.BlockSpec((1, H), lambda i: (0, 0)),
        ],
        out_specs=pl.BlockSpec((blk, H), lambda i: (i, 0)),
        out_shape=jax.ShapeDtypeStruct((rows, H), jnp.float32),
    )(v, w_row, b_row)


def _eft0(e4, sw, m, be_row, tb_row):
    def body(e_ref, sw_ref, m_ref, b_ref, tb_ref, ef_ref, t_ref):
        eb = e_ref[...]
        ef_ref[...] = jnp.dot(eb, sw_ref[...],
                              preferred_element_type=jnp.float32) + b_ref[...]
        t_ref[...] = jnp.dot(eb, m_ref[...],
                             preferred_element_type=jnp.float32) + tb_ref[...]

    spec = pl.BlockSpec((_BLK_P, 128), lambda i: (i, 0))
    row = pl.BlockSpec((1, 128), lambda i: (0, 0))
    return pl.pallas_call(
        body,
        grid=(E4 // _BLK_P,),
        in_specs=[pl.BlockSpec((_BLK_P, 4), lambda i: (i, 0)),
                  pl.BlockSpec((4, 128), lambda i: (0, 0)),
                  pl.BlockSpec((4, 128), lambda i: (0, 0)),
                  row, row],
        out_specs=[spec, spec],
        out_shape=[jax.ShapeDtypeStruct((E4, 128), jnp.float32)] * 2,
    )(e4, sw, m, be_row, tb_row)


def _eft(ef, eh, st, ge_row, be_row, a3blk, ba_row):
    def body(ef_ref, eh_ref, st_ref, g_ref, b_ref, a3_ref, ba_ref,
             ef_o, t_o):
        mu, inv = _stats_mean_var128(st_ref, float(E))
        bn = g_ref[...] * (eh_ref[...] - mu[None, :]) * inv[None, :] + b_ref[...]
        ef_new = ef_ref[...] + jnp.maximum(bn, 0.0)
        ef_o[...] = ef_new
        t_o[...] = jnp.dot(ef_new, a3_ref[...],
                           preferred_element_type=jnp.float32) + ba_ref[...]

    spec = pl.BlockSpec((_BLK_P, 128), lambda i: (i, 0))
    row = pl.BlockSpec((1, 128), lambda i: (0, 0))
    return pl.pallas_call(
        body,
        grid=(E4 // _BLK_P,),
        in_specs=[spec, spec,
                  pl.BlockSpec((NW, 64), lambda i: (0, 0)),
                  row, row,
                  pl.BlockSpec((128, 128), lambda i: (0, 0)), row],
        out_specs=[spec, spec],
        out_shape=[jax.ShapeDtypeStruct((E4, 128), jnp.float32)] * 2,
    )(ef, eh, st, ge_row, be_row, a3blk, ba_row)


def _node_pre(h, a1, a2, vv, u, bu_row):
    def body(h_ref, a1_ref, a2_ref, v_ref, u_ref, bu_ref,
             p1_ref, p2_ref, pv_ref, hu_ref):
        hb = h_ref[...]
        p1_ref[...] = jnp.dot(hb, a1_ref[...], preferred_element_type=jnp.float32)
        p2_ref[...] = jnp.dot(hb, a2_ref[...], preferred_element_type=jnp.float32)
        pv_ref[...] = jnp.dot(hb, v_ref[...], preferred_element_type=jnp.float32)
        hu_ref[...] = jnp.dot(hb, u_ref[...], preferred_element_type=jnp.float32) + bu_ref[...]

    spec = pl.BlockSpec((_BLK_N, H), lambda i: (i, 0))
    wspec = pl.BlockSpec((H, H), lambda i: (0, 0))
    return pl.pallas_call(
        body,
        grid=(N // _BLK_N,),
        in_specs=[spec, wspec, wspec, wspec, wspec,
                  pl.BlockSpec((1, H), lambda i: (0, 0))],
        out_specs=[spec, spec, spec, spec],
        out_shape=[jax.ShapeDtypeStruct((N, H), jnp.float32)] * 4,
    )(h, a1, a2, vv, u, bu_row)


def _node_hhat(hu, aggden):
    def body(hu_ref, agg_ref, den_ref, hh_ref, st_ref):
        hh = hu_ref[...] + agg_ref[...] / (den_ref[...] + 1e-6)
        hh_ref[...] = hh
        s1 = jnp.sum(hh, axis=0, keepdims=True)
        s2 = jnp.sum(hh * hh, axis=0, keepdims=True)
        blk_stats = jnp.concatenate(
            [s1, s2, jnp.zeros((6, H), jnp.float32)], axis=0)

        @pl.when(pl.program_id(0) == 0)
        def _():
            st_ref[...] = jnp.zeros((8, H), jnp.float32)

        st_ref[...] += blk_stats

    spec = pl.BlockSpec((_BLK_N, H), lambda i: (i, 0))
    return pl.pallas_call(
        body,
        grid=(N // _BLK_N,),
        in_specs=[spec, spec,
                  pl.BlockSpec((_BLK_N, H), lambda i: (i + N // _BLK_N, 0))],
        out_specs=[spec, pl.BlockSpec((8, H), lambda i: (0, 0))],
        out_shape=[
            jax.ShapeDtypeStruct((N, H), jnp.float32),
            jax.ShapeDtypeStruct((8, H), jnp.float32),
        ],
    )(hu, aggden, aggden)


def _node_apply(h, hh, st, gn_row, bnb_row):
    def body(h_ref, hh_ref, st_ref, g_ref, b_ref, o_ref):
        mu = st_ref[0, :] / N
        var = st_ref[1, :] / N - mu * mu
        inv = lax.rsqrt(var + 1e-5)
        bn = g_ref[...] * (hh_ref[...] - mu[None, :]) * inv[None, :] + b_ref[...]
        o_ref[...] = h_ref[...] + jnp.maximum(bn, 0.0)

    spec = pl.BlockSpec((_BLK_N, H), lambda i: (i, 0))
    return pl.pallas_call(
        body,
        grid=(N // _BLK_N,),
        in_specs=[spec, spec,
                  pl.BlockSpec((8, H), lambda i: (0, 0)),
                  pl.BlockSpec((1, H), lambda i: (0, 0)),
                  pl.BlockSpec((1, H), lambda i: (0, 0))],
        out_specs=spec,
        out_shape=jax.ShapeDtypeStruct((N, H), jnp.float32),
    )(h, hh, st, gn_row, bnb_row)


def _node_final(h, w1a, w1b):
    def body(h_ref, wa_ref, wb_ref, qs_ref, qd_ref):
        hb = h_ref[...]
        qs_ref[...] = jnp.dot(hb, wa_ref[...], preferred_element_type=jnp.float32)
        qd_ref[...] = jnp.dot(hb, wb_ref[...], preferred_element_type=jnp.float32)

    spec = pl.BlockSpec((_BLK_N, H), lambda i: (i, 0))
    wspec = pl.BlockSpec((H, H), lambda i: (0, 0))
    return pl.pallas_call(
        body,
        grid=(N // _BLK_N,),
        in_specs=[spec, wspec, wspec],
        out_specs=[spec, spec],
        out_shape=[jax.ShapeDtypeStruct((N, H), jnp.float32)] * 2,
    )(h, w1a, w1b)


def _final(gs, gd, ef, eh, st, ge_row, be_row, w1cblk, b1_row, w2p, b2_row):
    def body(gs_ref, gd_ref, ef_ref, eh_ref, st_ref, g_ref, b_ref,
             wc_ref, b1_ref, w2_ref, b2_ref, o_ref):
        mu, inv = _stats_mean_var128(st_ref, float(E))
        bn = g_ref[...] * (eh_ref[...] - mu[None, :]) * inv[None, :] + b_ref[...]
        ef_l = ef_ref[...] + jnp.maximum(bn, 0.0)
        t = jnp.dot(ef_l, wc_ref[...], preferred_element_type=jnp.float32)
        z1 = jnp.maximum(gs_ref[...] + gd_ref[...] + t + b1_ref[...], 0.0)
        o_ref[...] = jnp.dot(z1, w2_ref[...],
                             preferred_element_type=jnp.float32) + b2_ref[...]

    spec = pl.BlockSpec((_BLK_P, 128), lambda i: (i, 0))
    row = pl.BlockSpec((1, 128), lambda i: (0, 0))
    return pl.pallas_call(
        body,
        grid=(E4 // _BLK_P,),
        in_specs=[spec, spec, spec, spec,
                  pl.BlockSpec((NW, 64), lambda i: (0, 0)),
                  row, row,
                  pl.BlockSpec((128, 128), lambda i: (0, 0)), row,
                  pl.BlockSpec((128, 4), lambda i: (0, 0)),
                  pl.BlockSpec((1, 4), lambda i: (0, 0))],
        out_specs=pl.BlockSpec((_BLK_P, 4), lambda i: (i, 0)),
        out_shape=jax.ShapeDtypeStruct((E4, 4), jnp.float32),
    )(gs, gd, ef, eh, st, ge_row, be_row, w1cblk, b1_row, w2p, b2_row)


def kernel(x, e, edge_index, W_pe, b_pe, W_e, b_e, A1, A2, A3, U, V,
           bA, bU, gn, bnb, ge, be, W1, b1, W2, b2):
    src2d = edge_index[0].reshape(NCHUNKS, CH)
    dst2d = edge_index[1].reshape(NCHUNKS, CH)
    zeros = jnp.zeros((ROWS_PER_TILE, H), jnp.float32)
    eye4 = jnp.eye(4, dtype=jnp.float32)

    h = _bcast_init(x, W_pe, b_pe.reshape(1, H), N, _BLK_N)

    ef = None
    ehp = None
    st = None
    for l in range(L):
        a3blk = jnp.kron(eye4, A3[l])
        ba128 = jnp.tile(bA[l], 4).reshape(1, 128)
        if l == 0:
            sw = jnp.kron(eye4, W_e)
            m = jnp.kron(eye4, W_e @ A3[0])
            tbias = (jnp.tile(b_e, 4).reshape(1, 128) @ a3blk + ba128)
            ef, t = _eft0(e.reshape(E4, 4), sw, m,
                          jnp.tile(b_e, 4).reshape(1, 128), tbias)
        else:
            ef, t = _eft(ef, ehp, st, jnp.tile(ge[l - 1], 4).reshape(1, 128),
                         jnp.tile(be[l - 1], 4).reshape(1, 128), a3blk, ba128)
        p1, p2, pv, hu = _node_pre(h, A1[l], A2[l], V[l], U[l],
                                   bU[l].reshape(1, H))
        eh, sg, msg, st = _sc_edge(p1, p2, pv, t.reshape(E, H), dst2d, src2d)
        ehp = eh.reshape(E4, 128)
        aggden = _sc_scatter2(msg, sg, dst2d, zeros)
        hh, hst = _node_hhat(hu, aggden)
        h = _node_apply(h, hh, hst, gn[l].reshape(1, H), bnb[l].reshape(1, H))

    qs, qd = _node_final(h, W1[:H], W1[H:2 * H])
    gs, gd = _sc_gather2(qs, qd, dst2d, src2d)
    scores4 = _final(gs.reshape(E4, 128), gd.reshape(E4, 128), ef, ehp, st,
                     jnp.tile(ge[L - 1], 4).reshape(1, 128),
                     jnp.tile(be[L - 1], 4).reshape(1, 128),
                     jnp.kron(eye4, W1[2 * H:]), jnp.tile(b1, 4).reshape(1, 128),
                     jnp.kron(eye4, W2), jnp.tile(b2, 4).reshape(1, 4))
    return scores4.reshape(E, 1)
